# Initial kernel scaffold; baseline (speedup 1.0000x reference)
import jax
import jax.numpy as jnp
from jax.experimental import pallas as pl


def kernel(x, edge_index, edge_attr, u, batch, We1, be1, ge, bbe, We2, be2, Wn1, bn1, gn, bbn, Wn2, bn2, Wg1, bg1, gg, bbg, Wg2, bg2):
    raise NotImplementedError("write your pallas kernel here")


# speedup vs baseline: 4.6337x; 4.6337x over previous
import functools

import jax
import jax.numpy as jnp
from jax import lax
from jax.experimental import pallas as pl
from jax.experimental.pallas import tpu as pltpu
from jax.experimental.pallas import tpu_sc as plsc

_N = 10000
_E = 160000
_B = 16
_DN = 256
_DE = 16
_DG = 64
_H = 512
_EPS = 1e-5

_NPAD = 10240
_NB = 2048
_EB = 2000

_NW = 32
_GK = 40
_GPW = _E // _NW
_GCH = _GPW // _GK

_F32 = jnp.float32
_BF16 = jnp.bfloat16
_HH = _H // 2


def _pack_bf16_cols(ae, ao):
    be = lax.bitcast_convert_type(ae.astype(_BF16).astype(_F32), jnp.int32)
    bo = lax.bitcast_convert_type(ao.astype(_BF16).astype(_F32), jnp.int32)
    return lax.bitwise_or(lax.shift_right_logical(be, 16),
                          lax.bitwise_and(bo, jnp.int32(-65536)))


def _unpack_bf16_cols(g):
    he = lax.bitcast_convert_type(lax.shift_left(g, 16), _F32)
    ho = lax.bitcast_convert_type(lax.bitwise_and(g, jnp.int32(-65536)), _F32)
    return he, ho


def _nodeproj_body(x_ref, oh_ref, u_ref, w1se_ref, w1so_ref, w1de_ref,
                   w1do_ref, w1ue_ref, w1uo_ref, be1e_ref, be1o_ref,
                   p1_ref, p2_ref, ub_ref):
    oh = oh_ref[...]
    ub = jnp.dot(oh, u_ref[...], preferred_element_type=_F32)
    ub_ref[...] = ub
    x = x_ref[...]
    p1e = (jnp.dot(x, w1se_ref[...], preferred_element_type=_F32)
           + jnp.dot(ub, w1ue_ref[...], preferred_element_type=_F32)
           + be1e_ref[...])
    p1o = (jnp.dot(x, w1so_ref[...], preferred_element_type=_F32)
           + jnp.dot(ub, w1uo_ref[...], preferred_element_type=_F32)
           + be1o_ref[...])
    p1_ref[...] = _pack_bf16_cols(p1e, p1o)
    p2e = jnp.dot(x, w1de_ref[...], preferred_element_type=_F32)
    p2o = jnp.dot(x, w1do_ref[...], preferred_element_type=_F32)
    p2_ref[...] = _pack_bf16_cols(p2e, p2o)


def _node_projections(x, oh, u, w1se, w1so, w1de, w1do, w1ue, w1uo,
                      be1e, be1o):
    nblk = _NPAD // _NB
    cw = lambda shp: pl.BlockSpec(shp, lambda i: (0, 0))
    return pl.pallas_call(
        _nodeproj_body,
        grid=(nblk,),
        in_specs=[
            pl.BlockSpec((_NB, _DN), lambda i: (i, 0)),
            pl.BlockSpec((_NB, _B), lambda i: (i, 0)),
            cw((_B, _DG)),
            cw((_DN, _HH)), cw((_DN, _HH)), cw((_DN, _HH)), cw((_DN, _HH)),
            cw((_DG, _HH)), cw((_DG, _HH)),
            cw((1, _HH)), cw((1, _HH)),
        ],
        out_specs=[
            pl.BlockSpec((_NB, _HH), lambda i: (i, 0)),
            pl.BlockSpec((_NB, _HH), lambda i: (i, 0)),
            pl.BlockSpec((_NB, _DG), lambda i: (i, 0)),
        ],
        out_shape=[
            jax.ShapeDtypeStruct((_NPAD, _HH), jnp.int32),
            jax.ShapeDtypeStruct((_NPAD, _HH), jnp.int32),
            jax.ShapeDtypeStruct((_NPAD, _DG), _F32),
        ],
    )(x, oh, u, w1se, w1so, w1de, w1do, w1ue, w1uo, be1e, be1o)


def _sc_gather_combine(p1, p2, row, col):
    mesh = plsc.VectorSubcoreMesh(core_axis_name="c", subcore_axis_name="s")

    @functools.partial(
        pl.kernel,
        out_type=jax.ShapeDtypeStruct((_E, _HH), jnp.int32),
        mesh=mesh,
        compiler_params=pltpu.CompilerParams(needs_layout_passes=False),
        scratch_types=[
            pltpu.VMEM((_GPW,), jnp.int32),
            pltpu.VMEM((_GPW,), jnp.int32),
            pltpu.VMEM((_GK, _HH), jnp.int32),
            pltpu.VMEM((_GK, _HH), jnp.int32),
            pltpu.SemaphoreType.DMA,
            pltpu.SemaphoreType.DMA,
        ],
    )
    def k(p1_hbm, p2_hbm, row_hbm, col_hbm, g_hbm,
          ridx_all, cidx_all, bufa, bufb, sema, semb):
        wid = lax.axis_index("s") * 2 + lax.axis_index("c")
        base0 = wid * _GPW
        pltpu.sync_copy(row_hbm.at[pl.ds(base0, _GPW)], ridx_all)
        pltpu.sync_copy(col_hbm.at[pl.ds(base0, _GPW)], cidx_all)

        def chunk_body(i, carry):
            off = i * _GK
            da = pltpu.async_copy(
                p1_hbm.at[ridx_all.at[pl.ds(off, _GK)]], bufa, sema)
            db = pltpu.async_copy(
                p2_hbm.at[cidx_all.at[pl.ds(off, _GK)]], bufb, semb)
            da.wait()
            db.wait()

            def row_body(r, c2):
                for j in range(_HH // 16):
                    sl = pl.ds(j * 16, 16)
                    va = plsc.bitcast(bufa[r, sl], _BF16)
                    vb = plsc.bitcast(bufb[r, sl], _BF16)
                    bufa[r, sl] = plsc.bitcast(va + vb, jnp.int32)
                return c2

            lax.fori_loop(0, _GK, row_body, 0)
            pltpu.sync_copy(bufa, g_hbm.at[pl.ds(base0 + off, _GK)])
            return carry

        lax.fori_loop(0, _GCH, chunk_body, 0)

    return k(p1, p2, row, col)


def _estats_body(g_ref, ea_ref, w1ee_ref, w1eo_ref,
                 s1e_ref, s2e_ref, s1o_ref, s2o_ref):
    i = pl.program_id(0)
    ge, go = _unpack_bf16_cols(g_ref[...])
    ea = ea_ref[...]
    he = ge + jnp.dot(ea, w1ee_ref[...], preferred_element_type=_F32)
    ho = go + jnp.dot(ea, w1eo_ref[...], preferred_element_type=_F32)

    @pl.when(i == 0)
    def _init():
        s1e_ref[...] = jnp.zeros_like(s1e_ref)
        s2e_ref[...] = jnp.zeros_like(s2e_ref)
        s1o_ref[...] = jnp.zeros_like(s1o_ref)
        s2o_ref[...] = jnp.zeros_like(s2o_ref)

    s1e_ref[...] += jnp.sum(he, axis=0, keepdims=True)
    s2e_ref[...] += jnp.sum(he * he, axis=0, keepdims=True)
    s1o_ref[...] += jnp.sum(ho, axis=0, keepdims=True)
    s2o_ref[...] += jnp.sum(ho * ho, axis=0, keepdims=True)


def _edge_stats(g, ea, w1ee, w1eo):
    nblk = _E // _EB
    sspec = pl.BlockSpec((1, _HH), lambda i: (0, 0))
    return pl.pallas_call(
        _estats_body,
        grid=(nblk,),
        in_specs=[
            pl.BlockSpec((_EB, _HH), lambda i: (i, 0)),
            pl.BlockSpec((_EB, _DE), lambda i: (i, 0)),
            pl.BlockSpec((_DE, _HH), lambda i: (0, 0)),
            pl.BlockSpec((_DE, _HH), lambda i: (0, 0)),
        ],
        out_specs=[sspec, sspec, sspec, sspec],
        out_shape=[jax.ShapeDtypeStruct((1, _HH), _F32)] * 4,
    )(g, ea, w1ee, w1eo)


def _eout_body(g_ref, ea_ref, w1ee_ref, w1eo_ref, s1e_ref, s2e_ref,
               s1o_ref, s2o_ref, gee_ref, geo_ref, bbee_ref, bbeo_ref,
               w2e_ref, w2o_ref, be2_ref, be2t_ref, out_ref, outt_ref):
    ge_, go_ = _unpack_bf16_cols(g_ref[...])
    ea = ea_ref[...]
    he = ge_ + jnp.dot(ea, w1ee_ref[...], preferred_element_type=_F32)
    ho = go_ + jnp.dot(ea, w1eo_ref[...], preferred_element_type=_F32)

    def bn_relu(h, s1_, s2_, gam, bet):
        m = s1_ * (1.0 / _E)
        v = s2_ * (1.0 / _E) - m * m
        scale = lax.rsqrt(v + _EPS) * gam
        return jnp.maximum((h - m) * scale + bet, 0.0)

    re = bn_relu(he, s1e_ref[...], s2e_ref[...], gee_ref[...], bbee_ref[...])
    ro = bn_relu(ho, s1o_ref[...], s2o_ref[...], geo_ref[...], bbeo_ref[...])
    out_ref[...] = (jnp.dot(re, w2e_ref[...], preferred_element_type=_F32)
                    + jnp.dot(ro, w2o_ref[...], preferred_element_type=_F32)
                    + be2_ref[...])
    dnt = (((0,), (1,)), ((), ()))
    eot = (lax.dot_general(w2e_ref[...], re, dnt, preferred_element_type=_F32)
           + lax.dot_general(w2o_ref[...], ro, dnt,
                             preferred_element_type=_F32) + be2t_ref[...])
    outt_ref[...] = eot[None]


def _edge_out(g, ea, w1ee, w1eo, s1e, s2e, s1o, s2o, gee, geo, bbee, bbeo,
              w2e, w2o, be2, be2t):
    nblk = _E // _EB
    sspec = pl.BlockSpec((1, _HH), lambda i: (0, 0))
    return pl.pallas_call(
        _eout_body,
        grid=(nblk,),
        in_specs=[
            pl.BlockSpec((_EB, _HH), lambda i: (i, 0)),
            pl.BlockSpec((_EB, _DE), lambda i: (i, 0)),
            pl.BlockSpec((_DE, _HH), lambda i: (0, 0)),
            pl.BlockSpec((_DE, _HH), lambda i: (0, 0)),
            sspec, sspec, sspec, sspec, sspec, sspec, sspec, sspec,
            pl.BlockSpec((_HH, _DE), lambda i: (0, 0)),
            pl.BlockSpec((_HH, _DE), lambda i: (0, 0)),
            pl.BlockSpec((1, _DE), lambda i: (0, 0)),
            pl.BlockSpec((_DE, 1), lambda i: (0, 0)),
        ],
        out_specs=[
            pl.BlockSpec((_EB, _DE), lambda i: (i, 0)),
            pl.BlockSpec((1, _DE, _EB), lambda i: (i, 0, 0)),
        ],
        out_shape=[
            jax.ShapeDtypeStruct((_E, _DE), _F32),
            jax.ShapeDtypeStruct((nblk, _DE, _EB), _F32),
        ],
    )(g, ea, w1ee, w1eo, s1e, s2e, s1o, s2o, gee, geo, bbee, bbeo,
      w2e, w2o, be2, be2t)


def _sc_scatter_mean(eoutt, col):
    mesh = plsc.VectorSubcoreMesh(core_axis_name="c", subcore_axis_name="s")
    ngrp = _EB // 16
    nblk = _E // _EB

    @functools.partial(
        pl.kernel,
        out_type=(
            jax.ShapeDtypeStruct((2, _DE, _NPAD), _F32),
            jax.ShapeDtypeStruct((2, _DE, _NPAD), _F32),
        ),
        mesh=mesh,
        compiler_params=pltpu.CompilerParams(needs_layout_passes=False),
        scratch_types=[
            pltpu.VMEM((_EB,), jnp.int32),
            pltpu.VMEM((_EB,), jnp.int32),
            pltpu.VMEM((_EB,), _F32),
            pltpu.VMEM((_EB,), _F32),
            pltpu.SemaphoreType.DMA,
            pltpu.SemaphoreType.DMA,
            pltpu.SemaphoreType.DMA,
            pltpu.SemaphoreType.DMA,
            pltpu.VMEM((_NPAD,), _F32),
            pltpu.VMEM((_NPAD,), _F32),
        ],
    )
    def k(eoutt_hbm, col_hbm, sum_hbm, cnt_hbm,
          cidx0, cidx1, vals0, vals1, si0, si1, sv0, sv1,
          plane_s, plane_c):
        cid = lax.axis_index("c")
        sid = lax.axis_index("s")
        cbufs, vbufs = (cidx0, cidx1), (vals0, vals1)
        isems, vsems = (si0, si1), (sv0, sv1)
        nch = nblk // 2

        def fill_zero(r, c2):
            plane_s[pl.ds(r * 16, 16)] = jnp.zeros((16,), _F32)
            plane_c[pl.ds(r * 16, 16)] = jnp.zeros((16,), _F32)
            return c2

        lax.fori_loop(0, _NPAD // 16, fill_zero, 0)
        ones16 = jnp.full((16,), 1.0, _F32)

        def start(kk, s2):
            b = kk * 2 + cid
            pltpu.async_copy(col_hbm.at[pl.ds(b * _EB, _EB)],
                             cbufs[s2], isems[s2])
            pltpu.async_copy(eoutt_hbm.at[b, sid], vbufs[s2], vsems[s2])

        start(0, 0)

        def pair_body(pp, carry):
            for s2 in (0, 1):
                kk = pp * 2 + s2

                @pl.when(kk + 1 < nch)
                def _():
                    start(kk + 1, 1 - s2)

                b = kk * 2 + cid
                pltpu.make_async_copy(col_hbm.at[pl.ds(b * _EB, _EB)],
                                      cbufs[s2], isems[s2]).wait()
                pltpu.make_async_copy(eoutt_hbm.at[b, sid],
                                      vbufs[s2], vsems[s2]).wait()
                mine = lax.rem(kk, 16) == sid
                for j in range(ngrp):
                    idxv = cbufs[s2][pl.ds(j * 16, 16)]
                    vals = vbufs[s2][pl.ds(j * 16, 16)]
                    plsc.addupdate_scatter(plane_s, [idxv], vals)

                    @pl.when(mine)
                    def _():
                        plsc.addupdate_scatter(plane_c, [idxv], ones16)

            return carry

        lax.fori_loop(0, nch // 2, pair_body, 0)
        pltpu.sync_copy(plane_s, sum_hbm.at[cid, sid])
        pltpu.sync_copy(plane_c, cnt_hbm.at[cid, sid])

    return k(eoutt, col)


def _nstats_body(x_ref, s_ref, c_ref, ub_ref, wnx_ref, wne_ref, wnu_ref,
                 bn1_ref, hn_ref, s1_ref, s2_ref):
    i = pl.program_id(0)
    st = s_ref[0] + s_ref[1]
    c1 = jnp.sum(c_ref[0] + c_ref[1], axis=0, keepdims=True)
    eaggt = st / jnp.maximum(c1, 1.0)
    dn = (((0,), (0,)), ((), ()))
    h = (jnp.dot(x_ref[...], wnx_ref[...], preferred_element_type=_F32)
         + lax.dot_general(eaggt, wne_ref[...], dn, preferred_element_type=_F32)
         + jnp.dot(ub_ref[...], wnu_ref[...], preferred_element_type=_F32)
         + bn1_ref[...])
    hn_ref[...] = h
    rows = lax.broadcasted_iota(jnp.int32, (_NB, 1), 0) + i * _NB
    hm = jnp.where(rows < _N, h, 0.0)

    @pl.when(i == 0)
    def _init():
        s1_ref[...] = jnp.zeros_like(s1_ref)
        s2_ref[...] = jnp.zeros_like(s2_ref)

    s1_ref[...] += jnp.sum(hm, axis=0, keepdims=True)
    s2_ref[...] += jnp.sum(hm * hm, axis=0, keepdims=True)


def _node_stats(x, sums, cnts, ub, wnx, wne, wnu, bn1):
    nblk = _NPAD // _NB
    return pl.pallas_call(
        _nstats_body,
        grid=(nblk,),
        in_specs=[
            pl.BlockSpec((_NB, _DN), lambda i: (i, 0)),
            pl.BlockSpec((2, _DE, _NB), lambda i: (0, 0, i)),
            pl.BlockSpec((2, _DE, _NB), lambda i: (0, 0, i)),
            pl.BlockSpec((_NB, _DG), lambda i: (i, 0)),
            pl.BlockSpec((_DN, _H), lambda i: (0, 0)),
            pl.BlockSpec((_DE, _H), lambda i: (0, 0)),
            pl.BlockSpec((_DG, _H), lambda i: (0, 0)),
            pl.BlockSpec((1, _H), lambda i: (0, 0)),
        ],
        out_specs=[
            pl.BlockSpec((_NB, _H), lambda i: (i, 0)),
            pl.BlockSpec((1, _H), lambda i: (0, 0)),
            pl.BlockSpec((1, _H), lambda i: (0, 0)),
        ],
        out_shape=[
            jax.ShapeDtypeStruct((_NPAD, _H), _F32),
            jax.ShapeDtypeStruct((1, _H), _F32),
            jax.ShapeDtypeStruct((1, _H), _F32),
        ],
    )(x, sums, cnts, ub, wnx, wne, wnu, bn1)


def _nout_body(hn_ref, s1_ref, s2_ref, gn_ref, bbn_ref, wn2_ref, bn2_ref,
               oh_ref, s_ref, c_ref,
               xout_ref, xg_ref, nc_ref, eg_ref, ec_ref):
    i = pl.program_id(0)
    m = s1_ref[...] * (1.0 / _N)
    v = s2_ref[...] * (1.0 / _N) - m * m
    scale = lax.rsqrt(v + _EPS) * gn_ref[...]
    hn = (hn_ref[...] - m) * scale + bbn_ref[...]
    r = jnp.maximum(hn, 0.0)
    xo = jnp.dot(r, wn2_ref[...], preferred_element_type=_F32) + bn2_ref[...]
    xout_ref[...] = xo

    oh = oh_ref[...]
    st = s_ref[0] + s_ref[1]
    c1 = jnp.sum(c_ref[0] + c_ref[1], axis=0, keepdims=True)
    crep = jnp.broadcast_to(c1, (_DE, _NB))
    dn = (((0,), (0,)), ((), ()))
    dnt = (((0,), (1,)), ((), ()))

    @pl.when(i == 0)
    def _init():
        xg_ref[...] = jnp.zeros_like(xg_ref)
        nc_ref[...] = jnp.zeros_like(nc_ref)
        eg_ref[...] = jnp.zeros_like(eg_ref)
        ec_ref[...] = jnp.zeros_like(ec_ref)

    xg_ref[...] += lax.dot_general(oh, xo, dn, preferred_element_type=_F32)
    nc_ref[...] += lax.dot_general(oh, jnp.ones((_NB, _B), _F32), dn,
                                   preferred_element_type=_F32)
    eg_ref[...] += lax.dot_general(oh, st, dnt, preferred_element_type=_F32)
    ec_ref[...] += lax.dot_general(oh, crep, dnt, preferred_element_type=_F32)


def _node_out(hn, s1, s2, gn, bbn, wn2, bn2, oh, sums, cnts):
    nblk = _NPAD // _NB
    return pl.pallas_call(
        _nout_body,
        grid=(nblk,),
        in_specs=[
            pl.BlockSpec((_NB, _H), lambda i: (i, 0)),
            pl.BlockSpec((1, _H), lambda i: (0, 0)),
            pl.BlockSpec((1, _H), lambda i: (0, 0)),
            pl.BlockSpec((1, _H), lambda i: (0, 0)),
            pl.BlockSpec((1, _H), lambda i: (0, 0)),
            pl.BlockSpec((_H, _DN), lambda i: (0, 0)),
            pl.BlockSpec((1, _DN), lambda i: (0, 0)),
            pl.BlockSpec((_NB, _B), lambda i: (i, 0)),
            pl.BlockSpec((2, _DE, _NB), lambda i: (0, 0, i)),
            pl.BlockSpec((2, _DE, _NB), lambda i: (0, 0, i)),
        ],
        out_specs=[
            pl.BlockSpec((_NB, _DN), lambda i: (i, 0)),
            pl.BlockSpec((_B, _DN), lambda i: (0, 0)),
            pl.BlockSpec((_B, _B), lambda i: (0, 0)),
            pl.BlockSpec((_B, _DE), lambda i: (0, 0)),
            pl.BlockSpec((_B, _DE), lambda i: (0, 0)),
        ],
        out_shape=[
            jax.ShapeDtypeStruct((_NPAD, _DN), _F32),
            jax.ShapeDtypeStruct((_B, _DN), _F32),
            jax.ShapeDtypeStruct((_B, _B), _F32),
            jax.ShapeDtypeStruct((_B, _DE), _F32),
            jax.ShapeDtypeStruct((_B, _DE), _F32),
        ],
    )(hn, s1, s2, gn, bbn, wn2, bn2, oh, sums, cnts)


def _glob_body(u_ref, xg_ref, nc_ref, eg_ref, ec_ref, wgu_ref, wgx_ref,
               wge_ref, bg1_ref, gg_ref, bbg_ref, wg2_ref, bg2_ref, out_ref):
    ncnt = jnp.maximum(nc_ref[...][:, 0:1], 1.0)
    xag = xg_ref[...] / ncnt
    ecnt = jnp.maximum(ec_ref[...][:, 0:1], 1.0)
    eag = eg_ref[...] / ecnt
    h = (jnp.dot(u_ref[...], wgu_ref[...], preferred_element_type=_F32)
         + jnp.dot(xag, wgx_ref[...], preferred_element_type=_F32)
         + jnp.dot(eag, wge_ref[...], preferred_element_type=_F32)
         + bg1_ref[...])
    m = jnp.mean(h, axis=0, keepdims=True)
    v = jnp.mean((h - m) * (h - m), axis=0, keepdims=True)
    hn = (h - m) * (lax.rsqrt(v + _EPS) * gg_ref[...]) + bbg_ref[...]
    r = jnp.maximum(hn, 0.0)
    out_ref[...] = jnp.dot(r, wg2_ref[...], preferred_element_type=_F32) \
        + bg2_ref[...]


def _global_out(u, xg, nc, eg, ec, wgu, wgx, wge, bg1, gg, bbg, wg2, bg2):
    full = lambda s: pl.BlockSpec(s, lambda: tuple(0 for _ in s))
    return pl.pallas_call(
        _glob_body,
        in_specs=[
            full((_B, _DG)), full((_B, _DN)), full((_B, _B)),
            full((_B, _DE)), full((_B, _DE)),
            full((_DG, _H)), full((_DN, _H)), full((_DE, _H)),
            full((1, _H)), full((1, _H)), full((1, _H)),
            full((_H, _DG)), full((1, _DG)),
        ],
        out_specs=full((_B, _DG)),
        out_shape=jax.ShapeDtypeStruct((_B, _DG), _F32),
    )(u, xg, nc, eg, ec, wgu, wgx, wge, bg1, gg, bbg, wg2, bg2)


def kernel(x, edge_index, edge_attr, u, batch,
           We1, be1, ge, bbe, We2, be2,
           Wn1, bn1, gn, bbn, Wn2, bn2,
           Wg1, bg1, gg, bbg, Wg2, bg2):
    row = edge_index[0]
    col = edge_index[1]
    xp = jnp.pad(x, ((0, _NPAD - _N), (0, 0)))
    batchp = jnp.pad(batch, (0, _NPAD - _N), constant_values=-1)
    oh = (batchp[:, None] == jnp.arange(_B, dtype=batch.dtype)[None, :]
          ).astype(_F32)

    w1s = We1[:_DN]
    w1d = We1[_DN:2 * _DN]
    w1e = We1[2 * _DN:2 * _DN + _DE]
    w1u = We1[2 * _DN + _DE:]
    ev, od = slice(0, _H, 2), slice(1, _H, 2)
    wnx = Wn1[:_DN]
    wne = Wn1[_DN:_DN + _DE]
    wnu = Wn1[_DN + _DE:]
    wgu = Wg1[:_DG]
    wgx = Wg1[_DG:_DG + _DN]
    wge = Wg1[_DG + _DN:]

    p1, p2, ub = _node_projections(
        xp, oh, u, w1s[:, ev], w1s[:, od], w1d[:, ev], w1d[:, od],
        w1u[:, ev], w1u[:, od], be1[None, ev], be1[None, od])
    g = _sc_gather_combine(p1, p2, row, col)
    s1e, s2e, s1o, s2o = _edge_stats(g, edge_attr, w1e[:, ev], w1e[:, od])
    e_out, e_outt = _edge_out(
        g, edge_attr, w1e[:, ev], w1e[:, od], s1e, s2e, s1o, s2o,
        ge[None, ev], ge[None, od], bbe[None, ev], bbe[None, od],
        We2[ev], We2[od], be2[None, :], be2[:, None])
    sums, cnts = _sc_scatter_mean(e_outt, col)
    hn, t1, t2 = _node_stats(xp, sums, cnts, ub, wnx, wne, wnu, bn1[None, :])
    x_out, xg, nc, eg, ec = _node_out(hn, t1, t2, gn[None, :], bbn[None, :],
                                      Wn2, bn2[None, :], oh, sums, cnts)
    u_out = _global_out(u, xg, nc, eg, ec, wgu, wgx, wge, bg1[None, :],
                        gg[None, :], bbg[None, :], Wg2, bg2[None, :])
    return x_out[:_N], e_out, u_out


# speedup vs baseline: 5.4773x; 1.1821x over previous
import functools

import jax
import jax.numpy as jnp
from jax import lax
from jax.experimental import pallas as pl
from jax.experimental.pallas import tpu as pltpu
from jax.experimental.pallas import tpu_sc as plsc

_N = 10000
_E = 160000
_B = 16
_DN = 256
_DE = 16
_DG = 64
_H = 512
_EPS = 1e-5

_NPAD = 10240
_NB = 2048
_EB = 2000

_NW = 32
_GK = 40
_GPW = _E // _NW
_GCH = _GPW // _GK

_F32 = jnp.float32
_BF16 = jnp.bfloat16
_HH = _H // 2


def _pack_bf16_cols(ae, ao):
    be = lax.bitcast_convert_type(ae.astype(_BF16).astype(_F32), jnp.int32)
    bo = lax.bitcast_convert_type(ao.astype(_BF16).astype(_F32), jnp.int32)
    return lax.bitwise_or(lax.shift_right_logical(be, 16),
                          lax.bitwise_and(bo, jnp.int32(-65536)))


def _unpack_bf16_cols(g):
    he = lax.bitcast_convert_type(lax.shift_left(g, 16), _F32)
    ho = lax.bitcast_convert_type(lax.bitwise_and(g, jnp.int32(-65536)), _F32)
    return he, ho


def _nodeproj_body(x_ref, oh_ref, u_ref, w1se_ref, w1so_ref, w1de_ref,
                   w1do_ref, w1ue_ref, w1uo_ref, be1e_ref, be1o_ref,
                   p1_ref, p2_ref, ub_ref):
    oh = oh_ref[...]
    ub = jnp.dot(oh, u_ref[...], preferred_element_type=_F32)
    ub_ref[...] = ub
    x = x_ref[...]
    p1e = (jnp.dot(x, w1se_ref[...], preferred_element_type=_F32)
           + jnp.dot(ub, w1ue_ref[...], preferred_element_type=_F32)
           + be1e_ref[...])
    p1o = (jnp.dot(x, w1so_ref[...], preferred_element_type=_F32)
           + jnp.dot(ub, w1uo_ref[...], preferred_element_type=_F32)
           + be1o_ref[...])
    p1_ref[...] = _pack_bf16_cols(p1e, p1o)
    p2e = jnp.dot(x, w1de_ref[...], preferred_element_type=_F32)
    p2o = jnp.dot(x, w1do_ref[...], preferred_element_type=_F32)
    p2_ref[...] = _pack_bf16_cols(p2e, p2o)


def _node_projections(x, oh, u, w1se, w1so, w1de, w1do, w1ue, w1uo,
                      be1e, be1o):
    nblk = _NPAD // _NB
    cw = lambda shp: pl.BlockSpec(shp, lambda i: (0, 0))
    return pl.pallas_call(
        _nodeproj_body,
        grid=(nblk,),
        in_specs=[
            pl.BlockSpec((_NB, _DN), lambda i: (i, 0)),
            pl.BlockSpec((_NB, _B), lambda i: (i, 0)),
            cw((_B, _DG)),
            cw((_DN, _HH)), cw((_DN, _HH)), cw((_DN, _HH)), cw((_DN, _HH)),
            cw((_DG, _HH)), cw((_DG, _HH)),
            cw((1, _HH)), cw((1, _HH)),
        ],
        out_specs=[
            pl.BlockSpec((_NB, _HH), lambda i: (i, 0)),
            pl.BlockSpec((_NB, _HH), lambda i: (i, 0)),
            pl.BlockSpec((_NB, _DG), lambda i: (i, 0)),
        ],
        out_shape=[
            jax.ShapeDtypeStruct((_NPAD, _HH), jnp.int32),
            jax.ShapeDtypeStruct((_NPAD, _HH), jnp.int32),
            jax.ShapeDtypeStruct((_NPAD, _DG), _F32),
        ],
    )(x, oh, u, w1se, w1so, w1de, w1do, w1ue, w1uo, be1e, be1o)


def _sc_gather_combine(p1, p2, row, col):
    mesh = plsc.VectorSubcoreMesh(core_axis_name="c", subcore_axis_name="s")

    @functools.partial(
        pl.kernel,
        out_type=jax.ShapeDtypeStruct((_E, _HH), jnp.int32),
        mesh=mesh,
        compiler_params=pltpu.CompilerParams(needs_layout_passes=False),
        scratch_types=[
            pltpu.VMEM((_GPW,), jnp.int32),
            pltpu.VMEM((_GPW,), jnp.int32),
            pltpu.VMEM((_GK, _HH), jnp.int32),
            pltpu.VMEM((_GK, _HH), jnp.int32),
            pltpu.VMEM((_GK, _HH), jnp.int32),
            pltpu.VMEM((_GK, _HH), jnp.int32),
            pltpu.SemaphoreType.DMA,
            pltpu.SemaphoreType.DMA,
            pltpu.SemaphoreType.DMA,
            pltpu.SemaphoreType.DMA,
            pltpu.SemaphoreType.DMA,
            pltpu.SemaphoreType.DMA,
        ],
    )
    def k(p1_hbm, p2_hbm, row_hbm, col_hbm, g_hbm,
          ridx_all, cidx_all, bufa0, bufa1, bufb0, bufb1,
          sa0, sa1, sb0, sb1, so0, so1):
        wid = lax.axis_index("s") * 2 + lax.axis_index("c")
        base0 = wid * _GPW
        pltpu.sync_copy(row_hbm.at[pl.ds(base0, _GPW)], ridx_all)
        pltpu.sync_copy(col_hbm.at[pl.ds(base0, _GPW)], cidx_all)
        bufas, bufbs = (bufa0, bufa1), (bufb0, bufb1)
        sas, sbs, sos = (sa0, sa1), (sb0, sb1), (so0, so1)

        def start_gather(i, s2):
            off = i * _GK
            pltpu.async_copy(p1_hbm.at[ridx_all.at[pl.ds(off, _GK)]],
                             bufas[s2], sas[s2])
            pltpu.async_copy(p2_hbm.at[cidx_all.at[pl.ds(off, _GK)]],
                             bufbs[s2], sbs[s2])

        def wait_gather(i, s2):
            off = i * _GK
            pltpu.make_async_copy(p1_hbm.at[ridx_all.at[pl.ds(off, _GK)]],
                                  bufas[s2], sas[s2]).wait()
            pltpu.make_async_copy(p2_hbm.at[cidx_all.at[pl.ds(off, _GK)]],
                                  bufbs[s2], sbs[s2]).wait()

        def wait_out(i, s2):
            off = i * _GK
            pltpu.make_async_copy(bufas[s2],
                                  g_hbm.at[pl.ds(base0 + off, _GK)],
                                  sos[s2]).wait()

        start_gather(0, 0)

        def pair_body(pp, carry):
            for s2 in (0, 1):
                i = pp * 2 + s2

                @pl.when(i + 1 < _GCH)
                def _():
                    @pl.when(i >= 1)
                    def _():
                        wait_out(i - 1, 1 - s2)

                    start_gather(i + 1, 1 - s2)

                wait_gather(i, s2)
                buf_a, buf_b = bufas[s2], bufbs[s2]

                def row_body(r, c2):
                    for j in range(_HH // 16):
                        sl = pl.ds(j * 16, 16)
                        va = plsc.bitcast(buf_a[r, sl], _BF16)
                        vb = plsc.bitcast(buf_b[r, sl], _BF16)
                        buf_a[r, sl] = plsc.bitcast(va + vb, jnp.int32)
                    return c2

                lax.fori_loop(0, _GK, row_body, 0)
                off = i * _GK
                pltpu.async_copy(buf_a, g_hbm.at[pl.ds(base0 + off, _GK)],
                                 sos[s2])

            return carry

        lax.fori_loop(0, _GCH // 2, pair_body, 0)

        last = _GCH - 1
        wait_gather(last, 0)

        def row_body_l(r, c2):
            for j in range(_HH // 16):
                sl = pl.ds(j * 16, 16)
                va = plsc.bitcast(bufa0[r, sl], _BF16)
                vb = plsc.bitcast(bufb0[r, sl], _BF16)
                bufa0[r, sl] = plsc.bitcast(va + vb, jnp.int32)
            return c2

        lax.fori_loop(0, _GK, row_body_l, 0)
        pltpu.async_copy(bufa0, g_hbm.at[pl.ds(base0 + last * _GK, _GK)],
                         sos[0])
        wait_out(last - 1, 1)
        wait_out(last, 0)

    return k(p1, p2, row, col)


def _estats_body(g_ref, ea_ref, w1ee_ref, w1eo_ref,
                 s1e_ref, s2e_ref, s1o_ref, s2o_ref):
    i = pl.program_id(0)
    ge, go = _unpack_bf16_cols(g_ref[...])
    ea = ea_ref[...]
    he = ge + jnp.dot(ea, w1ee_ref[...], preferred_element_type=_F32)
    ho = go + jnp.dot(ea, w1eo_ref[...], preferred_element_type=_F32)

    @pl.when(i == 0)
    def _init():
        s1e_ref[...] = jnp.zeros_like(s1e_ref)
        s2e_ref[...] = jnp.zeros_like(s2e_ref)
        s1o_ref[...] = jnp.zeros_like(s1o_ref)
        s2o_ref[...] = jnp.zeros_like(s2o_ref)

    s1e_ref[...] += jnp.sum(he, axis=0, keepdims=True)
    s2e_ref[...] += jnp.sum(he * he, axis=0, keepdims=True)
    s1o_ref[...] += jnp.sum(ho, axis=0, keepdims=True)
    s2o_ref[...] += jnp.sum(ho * ho, axis=0, keepdims=True)


def _edge_stats(g, ea, w1ee, w1eo):
    nblk = _E // _EB
    sspec = pl.BlockSpec((1, _HH), lambda i: (0, 0))
    return pl.pallas_call(
        _estats_body,
        grid=(nblk,),
        in_specs=[
            pl.BlockSpec((_EB, _HH), lambda i: (i, 0)),
            pl.BlockSpec((_EB, _DE), lambda i: (i, 0)),
            pl.BlockSpec((_DE, _HH), lambda i: (0, 0)),
            pl.BlockSpec((_DE, _HH), lambda i: (0, 0)),
        ],
        out_specs=[sspec, sspec, sspec, sspec],
        out_shape=[jax.ShapeDtypeStruct((1, _HH), _F32)] * 4,
    )(g, ea, w1ee, w1eo)


def _eout_body(g_ref, ea_ref, w1ee_ref, w1eo_ref, s1e_ref, s2e_ref,
               s1o_ref, s2o_ref, gee_ref, geo_ref, bbee_ref, bbeo_ref,
               w2e_ref, w2o_ref, be2_ref, be2t_ref, out_ref, outt_ref):
    ge_, go_ = _unpack_bf16_cols(g_ref[...])
    ea = ea_ref[...]
    he = ge_ + jnp.dot(ea, w1ee_ref[...], preferred_element_type=_F32)
    ho = go_ + jnp.dot(ea, w1eo_ref[...], preferred_element_type=_F32)

    def bn_relu(h, s1_, s2_, gam, bet):
        m = s1_ * (1.0 / _E)
        v = s2_ * (1.0 / _E) - m * m
        scale = lax.rsqrt(v + _EPS) * gam
        return jnp.maximum((h - m) * scale + bet, 0.0)

    re = bn_relu(he, s1e_ref[...], s2e_ref[...], gee_ref[...], bbee_ref[...])
    ro = bn_relu(ho, s1o_ref[...], s2o_ref[...], geo_ref[...], bbeo_ref[...])
    out_ref[...] = (jnp.dot(re, w2e_ref[...], preferred_element_type=_F32)
                    + jnp.dot(ro, w2o_ref[...], preferred_element_type=_F32)
                    + be2_ref[...])
    dnt = (((0,), (1,)), ((), ()))
    eot = (lax.dot_general(w2e_ref[...], re, dnt, preferred_element_type=_F32)
           + lax.dot_general(w2o_ref[...], ro, dnt,
                             preferred_element_type=_F32) + be2t_ref[...])
    outt_ref[...] = eot[None]


def _edge_out(g, ea, w1ee, w1eo, s1e, s2e, s1o, s2o, gee, geo, bbee, bbeo,
              w2e, w2o, be2, be2t):
    nblk = _E // _EB
    sspec = pl.BlockSpec((1, _HH), lambda i: (0, 0))
    return pl.pallas_call(
        _eout_body,
        grid=(nblk,),
        in_specs=[
            pl.BlockSpec((_EB, _HH), lambda i: (i, 0)),
            pl.BlockSpec((_EB, _DE), lambda i: (i, 0)),
            pl.BlockSpec((_DE, _HH), lambda i: (0, 0)),
            pl.BlockSpec((_DE, _HH), lambda i: (0, 0)),
            sspec, sspec, sspec, sspec, sspec, sspec, sspec, sspec,
            pl.BlockSpec((_HH, _DE), lambda i: (0, 0)),
            pl.BlockSpec((_HH, _DE), lambda i: (0, 0)),
            pl.BlockSpec((1, _DE), lambda i: (0, 0)),
            pl.BlockSpec((_DE, 1), lambda i: (0, 0)),
        ],
        out_specs=[
            pl.BlockSpec((_EB, _DE), lambda i: (i, 0)),
            pl.BlockSpec((1, _DE, _EB), lambda i: (i, 0, 0)),
        ],
        out_shape=[
            jax.ShapeDtypeStruct((_E, _DE), _F32),
            jax.ShapeDtypeStruct((nblk, _DE, _EB), _F32),
        ],
    )(g, ea, w1ee, w1eo, s1e, s2e, s1o, s2o, gee, geo, bbee, bbeo,
      w2e, w2o, be2, be2t)


def _sc_scatter_mean(eoutt, col):
    mesh = plsc.VectorSubcoreMesh(core_axis_name="c", subcore_axis_name="s")
    ngrp = _EB // 16
    nblk = _E // _EB

    @functools.partial(
        pl.kernel,
        out_type=(
            jax.ShapeDtypeStruct((2, _DE, _NPAD), _F32),
            jax.ShapeDtypeStruct((2, _DE, _NPAD), _F32),
        ),
        mesh=mesh,
        compiler_params=pltpu.CompilerParams(needs_layout_passes=False),
        scratch_types=[
            pltpu.VMEM((_EB,), jnp.int32),
            pltpu.VMEM((_EB,), jnp.int32),
            pltpu.VMEM((_EB,), _F32),
            pltpu.VMEM((_EB,), _F32),
            pltpu.SemaphoreType.DMA,
            pltpu.SemaphoreType.DMA,
            pltpu.SemaphoreType.DMA,
            pltpu.SemaphoreType.DMA,
            pltpu.VMEM((_NPAD,), _F32),
            pltpu.VMEM((_NPAD,), _F32),
        ],
    )
    def k(eoutt_hbm, col_hbm, sum_hbm, cnt_hbm,
          cidx0, cidx1, vals0, vals1, si0, si1, sv0, sv1,
          plane_s, plane_c):
        cid = lax.axis_index("c")
        sid = lax.axis_index("s")
        cbufs, vbufs = (cidx0, cidx1), (vals0, vals1)
        isems, vsems = (si0, si1), (sv0, sv1)
        nch = nblk // 2

        def fill_zero(r, c2):
            plane_s[pl.ds(r * 16, 16)] = jnp.zeros((16,), _F32)
            plane_c[pl.ds(r * 16, 16)] = jnp.zeros((16,), _F32)
            return c2

        lax.fori_loop(0, _NPAD // 16, fill_zero, 0)
        ones16 = jnp.full((16,), 1.0, _F32)

        def start(kk, s2):
            b = kk * 2 + cid
            pltpu.async_copy(col_hbm.at[pl.ds(b * _EB, _EB)],
                             cbufs[s2], isems[s2])
            pltpu.async_copy(eoutt_hbm.at[b, sid], vbufs[s2], vsems[s2])

        start(0, 0)

        def pair_body(pp, carry):
            for s2 in (0, 1):
                kk = pp * 2 + s2

                @pl.when(kk + 1 < nch)
                def _():
                    start(kk + 1, 1 - s2)

                b = kk * 2 + cid
                pltpu.make_async_copy(col_hbm.at[pl.ds(b * _EB, _EB)],
                                      cbufs[s2], isems[s2]).wait()
                pltpu.make_async_copy(eoutt_hbm.at[b, sid],
                                      vbufs[s2], vsems[s2]).wait()
                mine = lax.rem(kk, 16) == sid
                for j in range(ngrp):
                    idxv = cbufs[s2][pl.ds(j * 16, 16)]
                    vals = vbufs[s2][pl.ds(j * 16, 16)]
                    plsc.addupdate_scatter(plane_s, [idxv], vals)

                    @pl.when(mine)
                    def _():
                        plsc.addupdate_scatter(plane_c, [idxv], ones16)

            return carry

        lax.fori_loop(0, nch // 2, pair_body, 0)
        pltpu.sync_copy(plane_s, sum_hbm.at[cid, sid])
        pltpu.sync_copy(plane_c, cnt_hbm.at[cid, sid])

    return k(eoutt, col)


def _nstats_body(x_ref, s_ref, c_ref, ub_ref, wnx_ref, wne_ref, wnu_ref,
                 bn1_ref, hn_ref, s1_ref, s2_ref):
    i = pl.program_id(0)
    st = s_ref[0] + s_ref[1]
    c1 = jnp.sum(c_ref[0] + c_ref[1], axis=0, keepdims=True)
    eaggt = st / jnp.maximum(c1, 1.0)
    dn = (((0,), (0,)), ((), ()))
    h = (jnp.dot(x_ref[...], wnx_ref[...], preferred_element_type=_F32)
         + lax.dot_general(eaggt, wne_ref[...], dn, preferred_element_type=_F32)
         + jnp.dot(ub_ref[...], wnu_ref[...], preferred_element_type=_F32)
         + bn1_ref[...])
    hn_ref[...] = h
    rows = lax.broadcasted_iota(jnp.int32, (_NB, 1), 0) + i * _NB
    hm = jnp.where(rows < _N, h, 0.0)

    @pl.when(i == 0)
    def _init():
        s1_ref[...] = jnp.zeros_like(s1_ref)
        s2_ref[...] = jnp.zeros_like(s2_ref)

    s1_ref[...] += jnp.sum(hm, axis=0, keepdims=True)
    s2_ref[...] += jnp.sum(hm * hm, axis=0, keepdims=True)


def _node_stats(x, sums, cnts, ub, wnx, wne, wnu, bn1):
    nblk = _NPAD // _NB
    return pl.pallas_call(
        _nstats_body,
        grid=(nblk,),
        in_specs=[
            pl.BlockSpec((_NB, _DN), lambda i: (i, 0)),
            pl.BlockSpec((2, _DE, _NB), lambda i: (0, 0, i)),
            pl.BlockSpec((2, _DE, _NB), lambda i: (0, 0, i)),
            pl.BlockSpec((_NB, _DG), lambda i: (i, 0)),
            pl.BlockSpec((_DN, _H), lambda i: (0, 0)),
            pl.BlockSpec((_DE, _H), lambda i: (0, 0)),
            pl.BlockSpec((_DG, _H), lambda i: (0, 0)),
            pl.BlockSpec((1, _H), lambda i: (0, 0)),
        ],
        out_specs=[
            pl.BlockSpec((_NB, _H), lambda i: (i, 0)),
            pl.BlockSpec((1, _H), lambda i: (0, 0)),
            pl.BlockSpec((1, _H), lambda i: (0, 0)),
        ],
        out_shape=[
            jax.ShapeDtypeStruct((_NPAD, _H), _F32),
            jax.ShapeDtypeStruct((1, _H), _F32),
            jax.ShapeDtypeStruct((1, _H), _F32),
        ],
    )(x, sums, cnts, ub, wnx, wne, wnu, bn1)


def _nout_body(hn_ref, s1_ref, s2_ref, gn_ref, bbn_ref, wn2_ref, bn2_ref,
               oh_ref, s_ref, c_ref,
               xout_ref, xg_ref, nc_ref, eg_ref, ec_ref):
    i = pl.program_id(0)
    m = s1_ref[...] * (1.0 / _N)
    v = s2_ref[...] * (1.0 / _N) - m * m
    scale = lax.rsqrt(v + _EPS) * gn_ref[...]
    hn = (hn_ref[...] - m) * scale + bbn_ref[...]
    r = jnp.maximum(hn, 0.0)
    xo = jnp.dot(r, wn2_ref[...], preferred_element_type=_F32) + bn2_ref[...]
    xout_ref[...] = xo

    oh = oh_ref[...]
    st = s_ref[0] + s_ref[1]
    c1 = jnp.sum(c_ref[0] + c_ref[1], axis=0, keepdims=True)
    crep = jnp.broadcast_to(c1, (_DE, _NB))
    dn = (((0,), (0,)), ((), ()))
    dnt = (((0,), (1,)), ((), ()))

    @pl.when(i == 0)
    def _init():
        xg_ref[...] = jnp.zeros_like(xg_ref)
        nc_ref[...] = jnp.zeros_like(nc_ref)
        eg_ref[...] = jnp.zeros_like(eg_ref)
        ec_ref[...] = jnp.zeros_like(ec_ref)

    xg_ref[...] += lax.dot_general(oh, xo, dn, preferred_element_type=_F32)
    nc_ref[...] += lax.dot_general(oh, jnp.ones((_NB, _B), _F32), dn,
                                   preferred_element_type=_F32)
    eg_ref[...] += lax.dot_general(oh, st, dnt, preferred_element_type=_F32)
    ec_ref[...] += lax.dot_general(oh, crep, dnt, preferred_element_type=_F32)


def _node_out(hn, s1, s2, gn, bbn, wn2, bn2, oh, sums, cnts):
    nblk = _NPAD // _NB
    return pl.pallas_call(
        _nout_body,
        grid=(nblk,),
        in_specs=[
            pl.BlockSpec((_NB, _H), lambda i: (i, 0)),
            pl.BlockSpec((1, _H), lambda i: (0, 0)),
            pl.BlockSpec((1, _H), lambda i: (0, 0)),
            pl.BlockSpec((1, _H), lambda i: (0, 0)),
            pl.BlockSpec((1, _H), lambda i: (0, 0)),
            pl.BlockSpec((_H, _DN), lambda i: (0, 0)),
            pl.BlockSpec((1, _DN), lambda i: (0, 0)),
            pl.BlockSpec((_NB, _B), lambda i: (i, 0)),
            pl.BlockSpec((2, _DE, _NB), lambda i: (0, 0, i)),
            pl.BlockSpec((2, _DE, _NB), lambda i: (0, 0, i)),
        ],
        out_specs=[
            pl.BlockSpec((_NB, _DN), lambda i: (i, 0)),
            pl.BlockSpec((_B, _DN), lambda i: (0, 0)),
            pl.BlockSpec((_B, _B), lambda i: (0, 0)),
            pl.BlockSpec((_B, _DE), lambda i: (0, 0)),
            pl.BlockSpec((_B, _DE), lambda i: (0, 0)),
        ],
        out_shape=[
            jax.ShapeDtypeStruct((_NPAD, _DN), _F32),
            jax.ShapeDtypeStruct((_B, _DN), _F32),
            jax.ShapeDtypeStruct((_B, _B), _F32),
            jax.ShapeDtypeStruct((_B, _DE), _F32),
            jax.ShapeDtypeStruct((_B, _DE), _F32),
        ],
    )(hn, s1, s2, gn, bbn, wn2, bn2, oh, sums, cnts)


def _glob_body(u_ref, xg_ref, nc_ref, eg_ref, ec_ref, wgu_ref, wgx_ref,
               wge_ref, bg1_ref, gg_ref, bbg_ref, wg2_ref, bg2_ref, out_ref):
    ncnt = jnp.maximum(nc_ref[...][:, 0:1], 1.0)
    xag = xg_ref[...] / ncnt
    ecnt = jnp.maximum(ec_ref[...][:, 0:1], 1.0)
    eag = eg_ref[...] / ecnt
    h = (jnp.dot(u_ref[...], wgu_ref[...], preferred_element_type=_F32)
         + jnp.dot(xag, wgx_ref[...], preferred_element_type=_F32)
         + jnp.dot(eag, wge_ref[...], preferred_element_type=_F32)
         + bg1_ref[...])
    m = jnp.mean(h, axis=0, keepdims=True)
    v = jnp.mean((h - m) * (h - m), axis=0, keepdims=True)
    hn = (h - m) * (lax.rsqrt(v + _EPS) * gg_ref[...]) + bbg_ref[...]
    r = jnp.maximum(hn, 0.0)
    out_ref[...] = jnp.dot(r, wg2_ref[...], preferred_element_type=_F32) \
        + bg2_ref[...]


def _global_out(u, xg, nc, eg, ec, wgu, wgx, wge, bg1, gg, bbg, wg2, bg2):
    full = lambda s: pl.BlockSpec(s, lambda: tuple(0 for _ in s))
    return pl.pallas_call(
        _glob_body,
        in_specs=[
            full((_B, _DG)), full((_B, _DN)), full((_B, _B)),
            full((_B, _DE)), full((_B, _DE)),
            full((_DG, _H)), full((_DN, _H)), full((_DE, _H)),
            full((1, _H)), full((1, _H)), full((1, _H)),
            full((_H, _DG)), full((1, _DG)),
        ],
        out_specs=full((_B, _DG)),
        out_shape=jax.ShapeDtypeStruct((_B, _DG), _F32),
    )(u, xg, nc, eg, ec, wgu, wgx, wge, bg1, gg, bbg, wg2, bg2)


def kernel(x, edge_index, edge_attr, u, batch,
           We1, be1, ge, bbe, We2, be2,
           Wn1, bn1, gn, bbn, Wn2, bn2,
           Wg1, bg1, gg, bbg, Wg2, bg2):
    row = edge_index[0]
    col = edge_index[1]
    xp = jnp.pad(x, ((0, _NPAD - _N), (0, 0)))
    batchp = jnp.pad(batch, (0, _NPAD - _N), constant_values=-1)
    oh = (batchp[:, None] == jnp.arange(_B, dtype=batch.dtype)[None, :]
          ).astype(_F32)

    w1s = We1[:_DN]
    w1d = We1[_DN:2 * _DN]
    w1e = We1[2 * _DN:2 * _DN + _DE]
    w1u = We1[2 * _DN + _DE:]
    ev, od = slice(0, _H, 2), slice(1, _H, 2)
    wnx = Wn1[:_DN]
    wne = Wn1[_DN:_DN + _DE]
    wnu = Wn1[_DN + _DE:]
    wgu = Wg1[:_DG]
    wgx = Wg1[_DG:_DG + _DN]
    wge = Wg1[_DG + _DN:]

    p1, p2, ub = _node_projections(
        xp, oh, u, w1s[:, ev], w1s[:, od], w1d[:, ev], w1d[:, od],
        w1u[:, ev], w1u[:, od], be1[None, ev], be1[None, od])
    g = _sc_gather_combine(p1, p2, row, col)
    s1e, s2e, s1o, s2o = _edge_stats(g, edge_attr, w1e[:, ev], w1e[:, od])
    e_out, e_outt = _edge_out(
        g, edge_attr, w1e[:, ev], w1e[:, od], s1e, s2e, s1o, s2o,
        ge[None, ev], ge[None, od], bbe[None, ev], bbe[None, od],
        We2[ev], We2[od], be2[None, :], be2[:, None])
    sums, cnts = _sc_scatter_mean(e_outt, col)
    hn, t1, t2 = _node_stats(xp, sums, cnts, ub, wnx, wne, wnu, bn1[None, :])
    x_out, xg, nc, eg, ec = _node_out(hn, t1, t2, gn[None, :], bbn[None, :],
                                      Wn2, bn2[None, :], oh, sums, cnts)
    u_out = _global_out(u, xg, nc, eg, ec, wgu, wgx, wge, bg1[None, :],
                        gg[None, :], bbg[None, :], Wg2, bg2[None, :])
    return x_out[:_N], e_out, u_out


# speedup vs baseline: 5.7451x; 1.0489x over previous
import functools

import jax
import jax.numpy as jnp
from jax import lax
from jax.experimental import pallas as pl
from jax.experimental.pallas import tpu as pltpu
from jax.experimental.pallas import tpu_sc as plsc

_N = 10000
_E = 160000
_B = 16
_DN = 256
_DE = 16
_DG = 64
_H = 512
_EPS = 1e-5

_NPAD = 10240
_NB = 2048
_EB = 4000

_NW = 32
_GK = 40
_GPW = _E // _NW
_GCH = _GPW // _GK

_F32 = jnp.float32
_BF16 = jnp.bfloat16
_HH = _H // 2


def _pack_bf16_cols(ae, ao):
    be = lax.bitcast_convert_type(ae.astype(_BF16).astype(_F32), jnp.int32)
    bo = lax.bitcast_convert_type(ao.astype(_BF16).astype(_F32), jnp.int32)
    return lax.bitwise_or(lax.shift_right_logical(be, 16),
                          lax.bitwise_and(bo, jnp.int32(-65536)))


def _unpack_bf16_cols(g):
    he = lax.bitcast_convert_type(lax.shift_left(g, 16), _F32)
    ho = lax.bitcast_convert_type(lax.bitwise_and(g, jnp.int32(-65536)), _F32)
    return he, ho


def _nodeproj_body(x_ref, oh_ref, u_ref, w1se_ref, w1so_ref, w1de_ref,
                   w1do_ref, w1ue_ref, w1uo_ref, be1e_ref, be1o_ref,
                   p1_ref, p2_ref, ub_ref):
    oh = oh_ref[...]
    ub = jnp.dot(oh, u_ref[...], preferred_element_type=_F32)
    ub_ref[...] = ub
    x = x_ref[...]
    p1e = (jnp.dot(x, w1se_ref[...], preferred_element_type=_F32)
           + jnp.dot(ub, w1ue_ref[...], preferred_element_type=_F32)
           + be1e_ref[...])
    p1o = (jnp.dot(x, w1so_ref[...], preferred_element_type=_F32)
           + jnp.dot(ub, w1uo_ref[...], preferred_element_type=_F32)
           + be1o_ref[...])
    p1_ref[...] = _pack_bf16_cols(p1e, p1o)
    p2e = jnp.dot(x, w1de_ref[...], preferred_element_type=_F32)
    p2o = jnp.dot(x, w1do_ref[...], preferred_element_type=_F32)
    p2_ref[...] = _pack_bf16_cols(p2e, p2o)


def _node_projections(x, oh, u, w1se, w1so, w1de, w1do, w1ue, w1uo,
                      be1e, be1o):
    nblk = _NPAD // _NB
    cw = lambda shp: pl.BlockSpec(shp, lambda i: (0, 0))
    return pl.pallas_call(
        _nodeproj_body,
        grid=(nblk,),
        in_specs=[
            pl.BlockSpec((_NB, _DN), lambda i: (i, 0)),
            pl.BlockSpec((_NB, _B), lambda i: (i, 0)),
            cw((_B, _DG)),
            cw((_DN, _HH)), cw((_DN, _HH)), cw((_DN, _HH)), cw((_DN, _HH)),
            cw((_DG, _HH)), cw((_DG, _HH)),
            cw((1, _HH)), cw((1, _HH)),
        ],
        out_specs=[
            pl.BlockSpec((_NB, _HH), lambda i: (i, 0)),
            pl.BlockSpec((_NB, _HH), lambda i: (i, 0)),
            pl.BlockSpec((_NB, _DG), lambda i: (i, 0)),
        ],
        out_shape=[
            jax.ShapeDtypeStruct((_NPAD, _HH), jnp.int32),
            jax.ShapeDtypeStruct((_NPAD, _HH), jnp.int32),
            jax.ShapeDtypeStruct((_NPAD, _DG), _F32),
        ],
    )(x, oh, u, w1se, w1so, w1de, w1do, w1ue, w1uo, be1e, be1o)


def _sc_gather_combine(p1, p2, row, col):
    mesh = plsc.VectorSubcoreMesh(core_axis_name="c", subcore_axis_name="s")

    @functools.partial(
        pl.kernel,
        out_type=jax.ShapeDtypeStruct((_E, _HH), jnp.int32),
        mesh=mesh,
        compiler_params=pltpu.CompilerParams(needs_layout_passes=False),
        scratch_types=[
            pltpu.VMEM((_GPW,), jnp.int32),
            pltpu.VMEM((_GPW,), jnp.int32),
            pltpu.VMEM((_GK, _HH), jnp.int32),
            pltpu.VMEM((_GK, _HH), jnp.int32),
            pltpu.VMEM((_GK, _HH), jnp.int32),
            pltpu.VMEM((_GK, _HH), jnp.int32),
            pltpu.SemaphoreType.DMA,
            pltpu.SemaphoreType.DMA,
            pltpu.SemaphoreType.DMA,
            pltpu.SemaphoreType.DMA,
            pltpu.SemaphoreType.DMA,
            pltpu.SemaphoreType.DMA,
        ],
    )
    def k(p1_hbm, p2_hbm, row_hbm, col_hbm, g_hbm,
          ridx_all, cidx_all, bufa0, bufa1, bufb0, bufb1,
          sa0, sa1, sb0, sb1, so0, so1):
        wid = lax.axis_index("s") * 2 + lax.axis_index("c")
        base0 = wid * _GPW
        pltpu.sync_copy(row_hbm.at[pl.ds(base0, _GPW)], ridx_all)
        pltpu.sync_copy(col_hbm.at[pl.ds(base0, _GPW)], cidx_all)
        bufas, bufbs = (bufa0, bufa1), (bufb0, bufb1)
        sas, sbs, sos = (sa0, sa1), (sb0, sb1), (so0, so1)

        def start_gather(i, s2):
            off = i * _GK
            pltpu.async_copy(p1_hbm.at[ridx_all.at[pl.ds(off, _GK)]],
                             bufas[s2], sas[s2])
            pltpu.async_copy(p2_hbm.at[cidx_all.at[pl.ds(off, _GK)]],
                             bufbs[s2], sbs[s2])

        def wait_gather(i, s2):
            off = i * _GK
            pltpu.make_async_copy(p1_hbm.at[ridx_all.at[pl.ds(off, _GK)]],
                                  bufas[s2], sas[s2]).wait()
            pltpu.make_async_copy(p2_hbm.at[cidx_all.at[pl.ds(off, _GK)]],
                                  bufbs[s2], sbs[s2]).wait()

        def wait_out(i, s2):
            off = i * _GK
            pltpu.make_async_copy(bufas[s2],
                                  g_hbm.at[pl.ds(base0 + off, _GK)],
                                  sos[s2]).wait()

        start_gather(0, 0)

        def pair_body(pp, carry):
            for s2 in (0, 1):
                i = pp * 2 + s2

                @pl.when(i + 1 < _GCH)
                def _():
                    @pl.when(i >= 1)
                    def _():
                        wait_out(i - 1, 1 - s2)

                    start_gather(i + 1, 1 - s2)

                wait_gather(i, s2)
                buf_a, buf_b = bufas[s2], bufbs[s2]

                def row_body(r, c2):
                    for j in range(_HH // 16):
                        sl = pl.ds(j * 16, 16)
                        va = plsc.bitcast(buf_a[r, sl], _BF16)
                        vb = plsc.bitcast(buf_b[r, sl], _BF16)
                        buf_a[r, sl] = plsc.bitcast(va + vb, jnp.int32)
                    return c2

                lax.fori_loop(0, _GK, row_body, 0)
                off = i * _GK
                pltpu.async_copy(buf_a, g_hbm.at[pl.ds(base0 + off, _GK)],
                                 sos[s2])

            return carry

        lax.fori_loop(0, _GCH // 2, pair_body, 0)

        last = _GCH - 1
        wait_gather(last, 0)

        def row_body_l(r, c2):
            for j in range(_HH // 16):
                sl = pl.ds(j * 16, 16)
                va = plsc.bitcast(bufa0[r, sl], _BF16)
                vb = plsc.bitcast(bufb0[r, sl], _BF16)
                bufa0[r, sl] = plsc.bitcast(va + vb, jnp.int32)
            return c2

        lax.fori_loop(0, _GK, row_body_l, 0)
        pltpu.async_copy(bufa0, g_hbm.at[pl.ds(base0 + last * _GK, _GK)],
                         sos[0])
        wait_out(last - 1, 1)
        wait_out(last, 0)

    return k(p1, p2, row, col)


def _estats_body(g_ref, ea_ref, w1ee_ref, w1eo_ref,
                 s1e_ref, s2e_ref, s1o_ref, s2o_ref):
    i = pl.program_id(0)
    ge, go = _unpack_bf16_cols(g_ref[...])
    ea = ea_ref[...]
    he = ge + jnp.dot(ea, w1ee_ref[...], preferred_element_type=_F32)
    ho = go + jnp.dot(ea, w1eo_ref[...], preferred_element_type=_F32)

    @pl.when(i == 0)
    def _init():
        s1e_ref[...] = jnp.zeros_like(s1e_ref)
        s2e_ref[...] = jnp.zeros_like(s2e_ref)
        s1o_ref[...] = jnp.zeros_like(s1o_ref)
        s2o_ref[...] = jnp.zeros_like(s2o_ref)

    s1e_ref[...] += jnp.sum(he, axis=0, keepdims=True)
    s2e_ref[...] += jnp.sum(he * he, axis=0, keepdims=True)
    s1o_ref[...] += jnp.sum(ho, axis=0, keepdims=True)
    s2o_ref[...] += jnp.sum(ho * ho, axis=0, keepdims=True)


def _edge_stats(g, ea, w1ee, w1eo):
    nblk = _E // _EB
    sspec = pl.BlockSpec((1, _HH), lambda i: (0, 0))
    return pl.pallas_call(
        _estats_body,
        grid=(nblk,),
        in_specs=[
            pl.BlockSpec((_EB, _HH), lambda i: (i, 0)),
            pl.BlockSpec((_EB, _DE), lambda i: (i, 0)),
            pl.BlockSpec((_DE, _HH), lambda i: (0, 0)),
            pl.BlockSpec((_DE, _HH), lambda i: (0, 0)),
        ],
        out_specs=[sspec, sspec, sspec, sspec],
        out_shape=[jax.ShapeDtypeStruct((1, _HH), _F32)] * 4,
    )(g, ea, w1ee, w1eo)


def _eout_body(g_ref, ea_ref, w1ee_ref, w1eo_ref, s1e_ref, s2e_ref,
               s1o_ref, s2o_ref, gee_ref, geo_ref, bbee_ref, bbeo_ref,
               w2e_ref, w2o_ref, be2_ref, be2t_ref, out_ref, outt_ref):
    ge_, go_ = _unpack_bf16_cols(g_ref[...])
    ea = ea_ref[...]
    he = ge_ + jnp.dot(ea, w1ee_ref[...], preferred_element_type=_F32)
    ho = go_ + jnp.dot(ea, w1eo_ref[...], preferred_element_type=_F32)

    def bn_relu(h, s1_, s2_, gam, bet):
        m = s1_ * (1.0 / _E)
        v = s2_ * (1.0 / _E) - m * m
        scale = lax.rsqrt(v + _EPS) * gam
        return jnp.maximum((h - m) * scale + bet, 0.0)

    re = bn_relu(he, s1e_ref[...], s2e_ref[...], gee_ref[...], bbee_ref[...])
    ro = bn_relu(ho, s1o_ref[...], s2o_ref[...], geo_ref[...], bbeo_ref[...])
    out_ref[...] = (jnp.dot(re, w2e_ref[...], preferred_element_type=_F32)
                    + jnp.dot(ro, w2o_ref[...], preferred_element_type=_F32)
                    + be2_ref[...])
    dnt = (((0,), (1,)), ((), ()))
    eot = (lax.dot_general(w2e_ref[...], re, dnt, preferred_element_type=_F32)
           + lax.dot_general(w2o_ref[...], ro, dnt,
                             preferred_element_type=_F32) + be2t_ref[...])
    outt_ref[...] = eot[None]


def _edge_out(g, ea, w1ee, w1eo, s1e, s2e, s1o, s2o, gee, geo, bbee, bbeo,
              w2e, w2o, be2, be2t):
    nblk = _E // _EB
    sspec = pl.BlockSpec((1, _HH), lambda i: (0, 0))
    return pl.pallas_call(
        _eout_body,
        grid=(nblk,),
        in_specs=[
            pl.BlockSpec((_EB, _HH), lambda i: (i, 0)),
            pl.BlockSpec((_EB, _DE), lambda i: (i, 0)),
            pl.BlockSpec((_DE, _HH), lambda i: (0, 0)),
            pl.BlockSpec((_DE, _HH), lambda i: (0, 0)),
            sspec, sspec, sspec, sspec, sspec, sspec, sspec, sspec,
            pl.BlockSpec((_HH, _DE), lambda i: (0, 0)),
            pl.BlockSpec((_HH, _DE), lambda i: (0, 0)),
            pl.BlockSpec((1, _DE), lambda i: (0, 0)),
            pl.BlockSpec((_DE, 1), lambda i: (0, 0)),
        ],
        out_specs=[
            pl.BlockSpec((_EB, _DE), lambda i: (i, 0)),
            pl.BlockSpec((1, _DE, _EB), lambda i: (i, 0, 0)),
        ],
        out_shape=[
            jax.ShapeDtypeStruct((_E, _DE), _F32),
            jax.ShapeDtypeStruct((nblk, _DE, _EB), _F32),
        ],
    )(g, ea, w1ee, w1eo, s1e, s2e, s1o, s2o, gee, geo, bbee, bbeo,
      w2e, w2o, be2, be2t)


def _sc_scatter_mean(eoutt, col):
    mesh = plsc.VectorSubcoreMesh(core_axis_name="c", subcore_axis_name="s")
    ngrp = _EB // 16
    nblk = _E // _EB

    @functools.partial(
        pl.kernel,
        out_type=(
            jax.ShapeDtypeStruct((2, _DE, _NPAD), _F32),
            jax.ShapeDtypeStruct((2, _DE, _NPAD), _F32),
        ),
        mesh=mesh,
        compiler_params=pltpu.CompilerParams(needs_layout_passes=False),
        scratch_types=[
            pltpu.VMEM((_EB,), jnp.int32),
            pltpu.VMEM((_EB,), jnp.int32),
            pltpu.VMEM((_EB,), _F32),
            pltpu.VMEM((_EB,), _F32),
            pltpu.SemaphoreType.DMA,
            pltpu.SemaphoreType.DMA,
            pltpu.SemaphoreType.DMA,
            pltpu.SemaphoreType.DMA,
            pltpu.VMEM((_NPAD,), _F32),
            pltpu.VMEM((_NPAD,), _F32),
        ],
    )
    def k(eoutt_hbm, col_hbm, sum_hbm, cnt_hbm,
          cidx0, cidx1, vals0, vals1, si0, si1, sv0, sv1,
          plane_s, plane_c):
        cid = lax.axis_index("c")
        sid = lax.axis_index("s")
        cbufs, vbufs = (cidx0, cidx1), (vals0, vals1)
        isems, vsems = (si0, si1), (sv0, sv1)
        nch = nblk // 2

        def fill_zero(r, c2):
            plane_s[pl.ds(r * 16, 16)] = jnp.zeros((16,), _F32)
            plane_c[pl.ds(r * 16, 16)] = jnp.zeros((16,), _F32)
            return c2

        lax.fori_loop(0, _NPAD // 16, fill_zero, 0)
        ones16 = jnp.full((16,), 1.0, _F32)

        def start(kk, s2):
            b = kk * 2 + cid
            pltpu.async_copy(col_hbm.at[pl.ds(b * _EB, _EB)],
                             cbufs[s2], isems[s2])
            pltpu.async_copy(eoutt_hbm.at[b, sid], vbufs[s2], vsems[s2])

        start(0, 0)

        def pair_body(pp, carry):
            for s2 in (0, 1):
                kk = pp * 2 + s2

                @pl.when(kk + 1 < nch)
                def _():
                    start(kk + 1, 1 - s2)

                b = kk * 2 + cid
                pltpu.make_async_copy(col_hbm.at[pl.ds(b * _EB, _EB)],
                                      cbufs[s2], isems[s2]).wait()
                pltpu.make_async_copy(eoutt_hbm.at[b, sid],
                                      vbufs[s2], vsems[s2]).wait()
                mine = lax.rem(kk, 16) == sid
                for j in range(ngrp):
                    idxv = cbufs[s2][pl.ds(j * 16, 16)]
                    vals = vbufs[s2][pl.ds(j * 16, 16)]
                    plsc.addupdate_scatter(plane_s, [idxv], vals)

                    @pl.when(mine)
                    def _():
                        plsc.addupdate_scatter(plane_c, [idxv], ones16)

            return carry

        lax.fori_loop(0, nch // 2, pair_body, 0)
        pltpu.sync_copy(plane_s, sum_hbm.at[cid, sid])
        pltpu.sync_copy(plane_c, cnt_hbm.at[cid, sid])

    return k(eoutt, col)


def _nstats_body(x_ref, s_ref, c_ref, ub_ref, wnx_ref, wne_ref, wnu_ref,
                 bn1_ref, hn_ref, s1_ref, s2_ref):
    i = pl.program_id(0)
    st = s_ref[0] + s_ref[1]
    c1 = jnp.sum(c_ref[0] + c_ref[1], axis=0, keepdims=True)
    eaggt = st / jnp.maximum(c1, 1.0)
    dn = (((0,), (0,)), ((), ()))
    h = (jnp.dot(x_ref[...], wnx_ref[...], preferred_element_type=_F32)
         + lax.dot_general(eaggt, wne_ref[...], dn, preferred_element_type=_F32)
         + jnp.dot(ub_ref[...], wnu_ref[...], preferred_element_type=_F32)
         + bn1_ref[...])
    hn_ref[...] = h
    rows = lax.broadcasted_iota(jnp.int32, (_NB, 1), 0) + i * _NB
    hm = jnp.where(rows < _N, h, 0.0)

    @pl.when(i == 0)
    def _init():
        s1_ref[...] = jnp.zeros_like(s1_ref)
        s2_ref[...] = jnp.zeros_like(s2_ref)

    s1_ref[...] += jnp.sum(hm, axis=0, keepdims=True)
    s2_ref[...] += jnp.sum(hm * hm, axis=0, keepdims=True)


def _node_stats(x, sums, cnts, ub, wnx, wne, wnu, bn1):
    nblk = _NPAD // _NB
    return pl.pallas_call(
        _nstats_body,
        grid=(nblk,),
        in_specs=[
            pl.BlockSpec((_NB, _DN), lambda i: (i, 0)),
            pl.BlockSpec((2, _DE, _NB), lambda i: (0, 0, i)),
            pl.BlockSpec((2, _DE, _NB), lambda i: (0, 0, i)),
            pl.BlockSpec((_NB, _DG), lambda i: (i, 0)),
            pl.BlockSpec((_DN, _H), lambda i: (0, 0)),
            pl.BlockSpec((_DE, _H), lambda i: (0, 0)),
            pl.BlockSpec((_DG, _H), lambda i: (0, 0)),
            pl.BlockSpec((1, _H), lambda i: (0, 0)),
        ],
        out_specs=[
            pl.BlockSpec((_NB, _H), lambda i: (i, 0)),
            pl.BlockSpec((1, _H), lambda i: (0, 0)),
            pl.BlockSpec((1, _H), lambda i: (0, 0)),
        ],
        out_shape=[
            jax.ShapeDtypeStruct((_NPAD, _H), _F32),
            jax.ShapeDtypeStruct((1, _H), _F32),
            jax.ShapeDtypeStruct((1, _H), _F32),
        ],
    )(x, sums, cnts, ub, wnx, wne, wnu, bn1)


def _nout_body(hn_ref, s1_ref, s2_ref, gn_ref, bbn_ref, wn2_ref, bn2_ref,
               oh_ref, s_ref, c_ref,
               xout_ref, xg_ref, nc_ref, eg_ref, ec_ref):
    i = pl.program_id(0)
    m = s1_ref[...] * (1.0 / _N)
    v = s2_ref[...] * (1.0 / _N) - m * m
    scale = lax.rsqrt(v + _EPS) * gn_ref[...]
    hn = (hn_ref[...] - m) * scale + bbn_ref[...]
    r = jnp.maximum(hn, 0.0)
    xo = jnp.dot(r, wn2_ref[...], preferred_element_type=_F32) + bn2_ref[...]
    xout_ref[...] = xo

    oh = oh_ref[...]
    st = s_ref[0] + s_ref[1]
    c1 = jnp.sum(c_ref[0] + c_ref[1], axis=0, keepdims=True)
    crep = jnp.broadcast_to(c1, (_DE, _NB))
    dn = (((0,), (0,)), ((), ()))
    dnt = (((0,), (1,)), ((), ()))

    @pl.when(i == 0)
    def _init():
        xg_ref[...] = jnp.zeros_like(xg_ref)
        nc_ref[...] = jnp.zeros_like(nc_ref)
        eg_ref[...] = jnp.zeros_like(eg_ref)
        ec_ref[...] = jnp.zeros_like(ec_ref)

    xg_ref[...] += lax.dot_general(oh, xo, dn, preferred_element_type=_F32)
    nc_ref[...] += lax.dot_general(oh, jnp.ones((_NB, _B), _F32), dn,
                                   preferred_element_type=_F32)
    eg_ref[...] += lax.dot_general(oh, st, dnt, preferred_element_type=_F32)
    ec_ref[...] += lax.dot_general(oh, crep, dnt, preferred_element_type=_F32)


def _node_out(hn, s1, s2, gn, bbn, wn2, bn2, oh, sums, cnts):
    nblk = _NPAD // _NB
    return pl.pallas_call(
        _nout_body,
        grid=(nblk,),
        in_specs=[
            pl.BlockSpec((_NB, _H), lambda i: (i, 0)),
            pl.BlockSpec((1, _H), lambda i: (0, 0)),
            pl.BlockSpec((1, _H), lambda i: (0, 0)),
            pl.BlockSpec((1, _H), lambda i: (0, 0)),
            pl.BlockSpec((1, _H), lambda i: (0, 0)),
            pl.BlockSpec((_H, _DN), lambda i: (0, 0)),
            pl.BlockSpec((1, _DN), lambda i: (0, 0)),
            pl.BlockSpec((_NB, _B), lambda i: (i, 0)),
            pl.BlockSpec((2, _DE, _NB), lambda i: (0, 0, i)),
            pl.BlockSpec((2, _DE, _NB), lambda i: (0, 0, i)),
        ],
        out_specs=[
            pl.BlockSpec((_NB, _DN), lambda i: (i, 0)),
            pl.BlockSpec((_B, _DN), lambda i: (0, 0)),
            pl.BlockSpec((_B, _B), lambda i: (0, 0)),
            pl.BlockSpec((_B, _DE), lambda i: (0, 0)),
            pl.BlockSpec((_B, _DE), lambda i: (0, 0)),
        ],
        out_shape=[
            jax.ShapeDtypeStruct((_NPAD, _DN), _F32),
            jax.ShapeDtypeStruct((_B, _DN), _F32),
            jax.ShapeDtypeStruct((_B, _B), _F32),
            jax.ShapeDtypeStruct((_B, _DE), _F32),
            jax.ShapeDtypeStruct((_B, _DE), _F32),
        ],
    )(hn, s1, s2, gn, bbn, wn2, bn2, oh, sums, cnts)


def _glob_body(u_ref, xg_ref, nc_ref, eg_ref, ec_ref, wgu_ref, wgx_ref,
               wge_ref, bg1_ref, gg_ref, bbg_ref, wg2_ref, bg2_ref, out_ref):
    ncnt = jnp.maximum(nc_ref[...][:, 0:1], 1.0)
    xag = xg_ref[...] / ncnt
    ecnt = jnp.maximum(ec_ref[...][:, 0:1], 1.0)
    eag = eg_ref[...] / ecnt
    h = (jnp.dot(u_ref[...], wgu_ref[...], preferred_element_type=_F32)
         + jnp.dot(xag, wgx_ref[...], preferred_element_type=_F32)
         + jnp.dot(eag, wge_ref[...], preferred_element_type=_F32)
         + bg1_ref[...])
    m = jnp.mean(h, axis=0, keepdims=True)
    v = jnp.mean((h - m) * (h - m), axis=0, keepdims=True)
    hn = (h - m) * (lax.rsqrt(v + _EPS) * gg_ref[...]) + bbg_ref[...]
    r = jnp.maximum(hn, 0.0)
    out_ref[...] = jnp.dot(r, wg2_ref[...], preferred_element_type=_F32) \
        + bg2_ref[...]


def _global_out(u, xg, nc, eg, ec, wgu, wgx, wge, bg1, gg, bbg, wg2, bg2):
    full = lambda s: pl.BlockSpec(s, lambda: tuple(0 for _ in s))
    return pl.pallas_call(
        _glob_body,
        in_specs=[
            full((_B, _DG)), full((_B, _DN)), full((_B, _B)),
            full((_B, _DE)), full((_B, _DE)),
            full((_DG, _H)), full((_DN, _H)), full((_DE, _H)),
            full((1, _H)), full((1, _H)), full((1, _H)),
            full((_H, _DG)), full((1, _DG)),
        ],
        out_specs=full((_B, _DG)),
        out_shape=jax.ShapeDtypeStruct((_B, _DG), _F32),
    )(u, xg, nc, eg, ec, wgu, wgx, wge, bg1, gg, bbg, wg2, bg2)


def kernel(x, edge_index, edge_attr, u, batch,
           We1, be1, ge, bbe, We2, be2,
           Wn1, bn1, gn, bbn, Wn2, bn2,
           Wg1, bg1, gg, bbg, Wg2, bg2):
    row = edge_index[0]
    col = edge_index[1]
    xp = jnp.pad(x, ((0, _NPAD - _N), (0, 0)))
    batchp = jnp.pad(batch, (0, _NPAD - _N), constant_values=-1)
    oh = (batchp[:, None] == jnp.arange(_B, dtype=batch.dtype)[None, :]
          ).astype(_F32)

    w1s = We1[:_DN]
    w1d = We1[_DN:2 * _DN]
    w1e = We1[2 * _DN:2 * _DN + _DE]
    w1u = We1[2 * _DN + _DE:]
    ev, od = slice(0, _H, 2), slice(1, _H, 2)
    wnx = Wn1[:_DN]
    wne = Wn1[_DN:_DN + _DE]
    wnu = Wn1[_DN + _DE:]
    wgu = Wg1[:_DG]
    wgx = Wg1[_DG:_DG + _DN]
    wge = Wg1[_DG + _DN:]

    p1, p2, ub = _node_projections(
        xp, oh, u, w1s[:, ev], w1s[:, od], w1d[:, ev], w1d[:, od],
        w1u[:, ev], w1u[:, od], be1[None, ev], be1[None, od])
    g = _sc_gather_combine(p1, p2, row, col)
    s1e, s2e, s1o, s2o = _edge_stats(g, edge_attr, w1e[:, ev], w1e[:, od])
    e_out, e_outt = _edge_out(
        g, edge_attr, w1e[:, ev], w1e[:, od], s1e, s2e, s1o, s2o,
        ge[None, ev], ge[None, od], bbe[None, ev], bbe[None, od],
        We2[ev], We2[od], be2[None, :], be2[:, None])
    sums, cnts = _sc_scatter_mean(e_outt, col)
    hn, t1, t2 = _node_stats(xp, sums, cnts, ub, wnx, wne, wnu, bn1[None, :])
    x_out, xg, nc, eg, ec = _node_out(hn, t1, t2, gn[None, :], bbn[None, :],
                                      Wn2, bn2[None, :], oh, sums, cnts)
    u_out = _global_out(u, xg, nc, eg, ec, wgu, wgx, wge, bg1[None, :],
                        gg[None, :], bbg[None, :], Wg2, bg2[None, :])
    return x_out[:_N], e_out, u_out


# speedup vs baseline: 5.8115x; 1.0116x over previous
import functools

import jax
import jax.numpy as jnp
from jax import lax
from jax.experimental import pallas as pl
from jax.experimental.pallas import tpu as pltpu
from jax.experimental.pallas import tpu_sc as plsc

_N = 10000
_E = 160000
_B = 16
_DN = 256
_DE = 16
_DG = 64
_H = 512
_EPS = 1e-5

_NPAD = 10240
_NB = 2048
_EB = 8000

_NW = 32
_GK = 40
_GPW = _E // _NW
_GCH = _GPW // _GK

_F32 = jnp.float32
_BF16 = jnp.bfloat16
_HH = _H // 2


def _pack_bf16_cols(ae, ao):
    be = lax.bitcast_convert_type(ae.astype(_BF16).astype(_F32), jnp.int32)
    bo = lax.bitcast_convert_type(ao.astype(_BF16).astype(_F32), jnp.int32)
    return lax.bitwise_or(lax.shift_right_logical(be, 16),
                          lax.bitwise_and(bo, jnp.int32(-65536)))


def _unpack_bf16_cols(g):
    he = lax.bitcast_convert_type(lax.shift_left(g, 16), _F32)
    ho = lax.bitcast_convert_type(lax.bitwise_and(g, jnp.int32(-65536)), _F32)
    return he, ho


def _nodeproj_body(x_ref, oh_ref, u_ref, w1se_ref, w1so_ref, w1de_ref,
                   w1do_ref, w1ue_ref, w1uo_ref, be1e_ref, be1o_ref,
                   p1_ref, p2_ref, ub_ref):
    oh = oh_ref[...]
    ub = jnp.dot(oh, u_ref[...], preferred_element_type=_F32)
    ub_ref[...] = ub
    x = x_ref[...]
    p1e = (jnp.dot(x, w1se_ref[...], preferred_element_type=_F32)
           + jnp.dot(ub, w1ue_ref[...], preferred_element_type=_F32)
           + be1e_ref[...])
    p1o = (jnp.dot(x, w1so_ref[...], preferred_element_type=_F32)
           + jnp.dot(ub, w1uo_ref[...], preferred_element_type=_F32)
           + be1o_ref[...])
    p1_ref[...] = _pack_bf16_cols(p1e, p1o)
    p2e = jnp.dot(x, w1de_ref[...], preferred_element_type=_F32)
    p2o = jnp.dot(x, w1do_ref[...], preferred_element_type=_F32)
    p2_ref[...] = _pack_bf16_cols(p2e, p2o)


def _node_projections(x, oh, u, w1se, w1so, w1de, w1do, w1ue, w1uo,
                      be1e, be1o):
    nblk = _NPAD // _NB
    cw = lambda shp: pl.BlockSpec(shp, lambda i: (0, 0))
    return pl.pallas_call(
        _nodeproj_body,
        grid=(nblk,),
        in_specs=[
            pl.BlockSpec((_NB, _DN), lambda i: (i, 0)),
            pl.BlockSpec((_NB, _B), lambda i: (i, 0)),
            cw((_B, _DG)),
            cw((_DN, _HH)), cw((_DN, _HH)), cw((_DN, _HH)), cw((_DN, _HH)),
            cw((_DG, _HH)), cw((_DG, _HH)),
            cw((1, _HH)), cw((1, _HH)),
        ],
        out_specs=[
            pl.BlockSpec((_NB, _HH), lambda i: (i, 0)),
            pl.BlockSpec((_NB, _HH), lambda i: (i, 0)),
            pl.BlockSpec((_NB, _DG), lambda i: (i, 0)),
        ],
        out_shape=[
            jax.ShapeDtypeStruct((_NPAD, _HH), jnp.int32),
            jax.ShapeDtypeStruct((_NPAD, _HH), jnp.int32),
            jax.ShapeDtypeStruct((_NPAD, _DG), _F32),
        ],
    )(x, oh, u, w1se, w1so, w1de, w1do, w1ue, w1uo, be1e, be1o)


def _sc_gather_combine(p1, p2, row, col):
    mesh = plsc.VectorSubcoreMesh(core_axis_name="c", subcore_axis_name="s")

    @functools.partial(
        pl.kernel,
        out_type=jax.ShapeDtypeStruct((_E, _HH), jnp.int32),
        mesh=mesh,
        compiler_params=pltpu.CompilerParams(needs_layout_passes=False),
        scratch_types=[
            pltpu.VMEM((_GPW,), jnp.int32),
            pltpu.VMEM((_GPW,), jnp.int32),
            pltpu.VMEM((_GK, _HH), jnp.int32),
            pltpu.VMEM((_GK, _HH), jnp.int32),
            pltpu.VMEM((_GK, _HH), jnp.int32),
            pltpu.VMEM((_GK, _HH), jnp.int32),
            pltpu.SemaphoreType.DMA,
            pltpu.SemaphoreType.DMA,
            pltpu.SemaphoreType.DMA,
            pltpu.SemaphoreType.DMA,
            pltpu.SemaphoreType.DMA,
            pltpu.SemaphoreType.DMA,
        ],
    )
    def k(p1_hbm, p2_hbm, row_hbm, col_hbm, g_hbm,
          ridx_all, cidx_all, bufa0, bufa1, bufb0, bufb1,
          sa0, sa1, sb0, sb1, so0, so1):
        wid = lax.axis_index("s") * 2 + lax.axis_index("c")
        base0 = wid * _GPW
        pltpu.sync_copy(row_hbm.at[pl.ds(base0, _GPW)], ridx_all)
        pltpu.sync_copy(col_hbm.at[pl.ds(base0, _GPW)], cidx_all)
        bufas, bufbs = (bufa0, bufa1), (bufb0, bufb1)
        sas, sbs, sos = (sa0, sa1), (sb0, sb1), (so0, so1)

        def start_gather(i, s2):
            off = i * _GK
            pltpu.async_copy(p1_hbm.at[ridx_all.at[pl.ds(off, _GK)]],
                             bufas[s2], sas[s2])
            pltpu.async_copy(p2_hbm.at[cidx_all.at[pl.ds(off, _GK)]],
                             bufbs[s2], sbs[s2])

        def wait_gather(i, s2):
            off = i * _GK
            pltpu.make_async_copy(p1_hbm.at[ridx_all.at[pl.ds(off, _GK)]],
                                  bufas[s2], sas[s2]).wait()
            pltpu.make_async_copy(p2_hbm.at[cidx_all.at[pl.ds(off, _GK)]],
                                  bufbs[s2], sbs[s2]).wait()

        def wait_out(i, s2):
            off = i * _GK
            pltpu.make_async_copy(bufas[s2],
                                  g_hbm.at[pl.ds(base0 + off, _GK)],
                                  sos[s2]).wait()

        start_gather(0, 0)

        def pair_body(pp, carry):
            for s2 in (0, 1):
                i = pp * 2 + s2

                @pl.when(i + 1 < _GCH)
                def _():
                    @pl.when(i >= 1)
                    def _():
                        wait_out(i - 1, 1 - s2)

                    start_gather(i + 1, 1 - s2)

                wait_gather(i, s2)
                buf_a, buf_b = bufas[s2], bufbs[s2]

                def row_body(r, c2):
                    for j in range(_HH // 16):
                        sl = pl.ds(j * 16, 16)
                        va = plsc.bitcast(buf_a[r, sl], _BF16)
                        vb = plsc.bitcast(buf_b[r, sl], _BF16)
                        buf_a[r, sl] = plsc.bitcast(va + vb, jnp.int32)
                    return c2

                lax.fori_loop(0, _GK, row_body, 0)
                off = i * _GK
                pltpu.async_copy(buf_a, g_hbm.at[pl.ds(base0 + off, _GK)],
                                 sos[s2])

            return carry

        lax.fori_loop(0, _GCH // 2, pair_body, 0)

        last = _GCH - 1
        wait_gather(last, 0)

        def row_body_l(r, c2):
            for j in range(_HH // 16):
                sl = pl.ds(j * 16, 16)
                va = plsc.bitcast(bufa0[r, sl], _BF16)
                vb = plsc.bitcast(bufb0[r, sl], _BF16)
                bufa0[r, sl] = plsc.bitcast(va + vb, jnp.int32)
            return c2

        lax.fori_loop(0, _GK, row_body_l, 0)
        pltpu.async_copy(bufa0, g_hbm.at[pl.ds(base0 + last * _GK, _GK)],
                         sos[0])
        wait_out(last - 1, 1)
        wait_out(last, 0)

    return k(p1, p2, row, col)


def _estats_body(g_ref, ea_ref, w1ee_ref, w1eo_ref,
                 s1e_ref, s2e_ref, s1o_ref, s2o_ref):
    i = pl.program_id(0)
    ge, go = _unpack_bf16_cols(g_ref[...])
    ea = ea_ref[...]
    he = ge + jnp.dot(ea, w1ee_ref[...], preferred_element_type=_F32)
    ho = go + jnp.dot(ea, w1eo_ref[...], preferred_element_type=_F32)

    @pl.when(i == 0)
    def _init():
        s1e_ref[...] = jnp.zeros_like(s1e_ref)
        s2e_ref[...] = jnp.zeros_like(s2e_ref)
        s1o_ref[...] = jnp.zeros_like(s1o_ref)
        s2o_ref[...] = jnp.zeros_like(s2o_ref)

    s1e_ref[...] += jnp.sum(he, axis=0, keepdims=True)
    s2e_ref[...] += jnp.sum(he * he, axis=0, keepdims=True)
    s1o_ref[...] += jnp.sum(ho, axis=0, keepdims=True)
    s2o_ref[...] += jnp.sum(ho * ho, axis=0, keepdims=True)


def _edge_stats(g, ea, w1ee, w1eo):
    nblk = _E // _EB
    sspec = pl.BlockSpec((1, _HH), lambda i: (0, 0))
    return pl.pallas_call(
        _estats_body,
        grid=(nblk,),
        in_specs=[
            pl.BlockSpec((_EB, _HH), lambda i: (i, 0)),
            pl.BlockSpec((_EB, _DE), lambda i: (i, 0)),
            pl.BlockSpec((_DE, _HH), lambda i: (0, 0)),
            pl.BlockSpec((_DE, _HH), lambda i: (0, 0)),
        ],
        out_specs=[sspec, sspec, sspec, sspec],
        out_shape=[jax.ShapeDtypeStruct((1, _HH), _F32)] * 4,
    )(g, ea, w1ee, w1eo)


def _eout_body(g_ref, ea_ref, w1ee_ref, w1eo_ref, s1e_ref, s2e_ref,
               s1o_ref, s2o_ref, gee_ref, geo_ref, bbee_ref, bbeo_ref,
               w2e_ref, w2o_ref, be2_ref, be2t_ref, out_ref, outt_ref):
    ge_, go_ = _unpack_bf16_cols(g_ref[...])
    ea = ea_ref[...]
    he = ge_ + jnp.dot(ea, w1ee_ref[...], preferred_element_type=_F32)
    ho = go_ + jnp.dot(ea, w1eo_ref[...], preferred_element_type=_F32)

    def bn_relu(h, s1_, s2_, gam, bet):
        m = s1_ * (1.0 / _E)
        v = s2_ * (1.0 / _E) - m * m
        scale = lax.rsqrt(v + _EPS) * gam
        return jnp.maximum((h - m) * scale + bet, 0.0)

    re = bn_relu(he, s1e_ref[...], s2e_ref[...], gee_ref[...], bbee_ref[...])
    ro = bn_relu(ho, s1o_ref[...], s2o_ref[...], geo_ref[...], bbeo_ref[...])
    out_ref[...] = (jnp.dot(re, w2e_ref[...], preferred_element_type=_F32)
                    + jnp.dot(ro, w2o_ref[...], preferred_element_type=_F32)
                    + be2_ref[...])
    dnt = (((0,), (1,)), ((), ()))
    eot = (lax.dot_general(w2e_ref[...], re, dnt, preferred_element_type=_F32)
           + lax.dot_general(w2o_ref[...], ro, dnt,
                             preferred_element_type=_F32) + be2t_ref[...])
    outt_ref[...] = eot[None]


def _edge_out(g, ea, w1ee, w1eo, s1e, s2e, s1o, s2o, gee, geo, bbee, bbeo,
              w2e, w2o, be2, be2t):
    nblk = _E // _EB
    sspec = pl.BlockSpec((1, _HH), lambda i: (0, 0))
    return pl.pallas_call(
        _eout_body,
        grid=(nblk,),
        in_specs=[
            pl.BlockSpec((_EB, _HH), lambda i: (i, 0)),
            pl.BlockSpec((_EB, _DE), lambda i: (i, 0)),
            pl.BlockSpec((_DE, _HH), lambda i: (0, 0)),
            pl.BlockSpec((_DE, _HH), lambda i: (0, 0)),
            sspec, sspec, sspec, sspec, sspec, sspec, sspec, sspec,
            pl.BlockSpec((_HH, _DE), lambda i: (0, 0)),
            pl.BlockSpec((_HH, _DE), lambda i: (0, 0)),
            pl.BlockSpec((1, _DE), lambda i: (0, 0)),
            pl.BlockSpec((_DE, 1), lambda i: (0, 0)),
        ],
        out_specs=[
            pl.BlockSpec((_EB, _DE), lambda i: (i, 0)),
            pl.BlockSpec((1, _DE, _EB), lambda i: (i, 0, 0)),
        ],
        out_shape=[
            jax.ShapeDtypeStruct((_E, _DE), _F32),
            jax.ShapeDtypeStruct((nblk, _DE, _EB), _F32),
        ],
    )(g, ea, w1ee, w1eo, s1e, s2e, s1o, s2o, gee, geo, bbee, bbeo,
      w2e, w2o, be2, be2t)


def _sc_scatter_mean(eoutt, col):
    mesh = plsc.VectorSubcoreMesh(core_axis_name="c", subcore_axis_name="s")
    ngrp = _EB // 16
    nblk = _E // _EB

    @functools.partial(
        pl.kernel,
        out_type=(
            jax.ShapeDtypeStruct((2, _DE, _NPAD), _F32),
            jax.ShapeDtypeStruct((2, _DE, _NPAD), _F32),
        ),
        mesh=mesh,
        compiler_params=pltpu.CompilerParams(needs_layout_passes=False),
        scratch_types=[
            pltpu.VMEM((_EB,), jnp.int32),
            pltpu.VMEM((_EB,), jnp.int32),
            pltpu.VMEM((_EB,), _F32),
            pltpu.VMEM((_EB,), _F32),
            pltpu.SemaphoreType.DMA,
            pltpu.SemaphoreType.DMA,
            pltpu.SemaphoreType.DMA,
            pltpu.SemaphoreType.DMA,
            pltpu.VMEM((_NPAD,), _F32),
            pltpu.VMEM((_NPAD,), _F32),
        ],
    )
    def k(eoutt_hbm, col_hbm, sum_hbm, cnt_hbm,
          cidx0, cidx1, vals0, vals1, si0, si1, sv0, sv1,
          plane_s, plane_c):
        cid = lax.axis_index("c")
        sid = lax.axis_index("s")
        cbufs, vbufs = (cidx0, cidx1), (vals0, vals1)
        isems, vsems = (si0, si1), (sv0, sv1)
        nch = nblk // 2

        def fill_zero(r, c2):
            plane_s[pl.ds(r * 16, 16)] = jnp.zeros((16,), _F32)
            plane_c[pl.ds(r * 16, 16)] = jnp.zeros((16,), _F32)
            return c2

        lax.fori_loop(0, _NPAD // 16, fill_zero, 0)
        ones16 = jnp.full((16,), 1.0, _F32)

        def start(kk, s2):
            b = kk * 2 + cid
            pltpu.async_copy(col_hbm.at[pl.ds(b * _EB, _EB)],
                             cbufs[s2], isems[s2])
            pltpu.async_copy(eoutt_hbm.at[b, sid], vbufs[s2], vsems[s2])

        start(0, 0)

        def pair_body(pp, carry):
            for s2 in (0, 1):
                kk = pp * 2 + s2

                @pl.when(kk + 1 < nch)
                def _():
                    start(kk + 1, 1 - s2)

                b = kk * 2 + cid
                pltpu.make_async_copy(col_hbm.at[pl.ds(b * _EB, _EB)],
                                      cbufs[s2], isems[s2]).wait()
                pltpu.make_async_copy(eoutt_hbm.at[b, sid],
                                      vbufs[s2], vsems[s2]).wait()
                mine = lax.rem(kk, 16) == sid
                for j in range(ngrp):
                    idxv = cbufs[s2][pl.ds(j * 16, 16)]
                    vals = vbufs[s2][pl.ds(j * 16, 16)]
                    plsc.addupdate_scatter(plane_s, [idxv], vals)

                    @pl.when(mine)
                    def _():
                        plsc.addupdate_scatter(plane_c, [idxv], ones16)

            return carry

        lax.fori_loop(0, nch // 2, pair_body, 0)
        pltpu.sync_copy(plane_s, sum_hbm.at[cid, sid])
        pltpu.sync_copy(plane_c, cnt_hbm.at[cid, sid])

    return k(eoutt, col)


def _nstats_body(x_ref, s_ref, c_ref, ub_ref, wnx_ref, wne_ref, wnu_ref,
                 bn1_ref, hn_ref, s1_ref, s2_ref):
    i = pl.program_id(0)
    st = s_ref[0] + s_ref[1]
    c1 = jnp.sum(c_ref[0] + c_ref[1], axis=0, keepdims=True)
    eaggt = st / jnp.maximum(c1, 1.0)
    dn = (((0,), (0,)), ((), ()))
    h = (jnp.dot(x_ref[...], wnx_ref[...], preferred_element_type=_F32)
         + lax.dot_general(eaggt, wne_ref[...], dn, preferred_element_type=_F32)
         + jnp.dot(ub_ref[...], wnu_ref[...], preferred_element_type=_F32)
         + bn1_ref[...])
    hn_ref[...] = h
    rows = lax.broadcasted_iota(jnp.int32, (_NB, 1), 0) + i * _NB
    hm = jnp.where(rows < _N, h, 0.0)

    @pl.when(i == 0)
    def _init():
        s1_ref[...] = jnp.zeros_like(s1_ref)
        s2_ref[...] = jnp.zeros_like(s2_ref)

    s1_ref[...] += jnp.sum(hm, axis=0, keepdims=True)
    s2_ref[...] += jnp.sum(hm * hm, axis=0, keepdims=True)


def _node_stats(x, sums, cnts, ub, wnx, wne, wnu, bn1):
    nblk = _NPAD // _NB
    return pl.pallas_call(
        _nstats_body,
        grid=(nblk,),
        in_specs=[
            pl.BlockSpec((_NB, _DN), lambda i: (i, 0)),
            pl.BlockSpec((2, _DE, _NB), lambda i: (0, 0, i)),
            pl.BlockSpec((2, _DE, _NB), lambda i: (0, 0, i)),
            pl.BlockSpec((_NB, _DG), lambda i: (i, 0)),
            pl.BlockSpec((_DN, _H), lambda i: (0, 0)),
            pl.BlockSpec((_DE, _H), lambda i: (0, 0)),
            pl.BlockSpec((_DG, _H), lambda i: (0, 0)),
            pl.BlockSpec((1, _H), lambda i: (0, 0)),
        ],
        out_specs=[
            pl.BlockSpec((_NB, _H), lambda i: (i, 0)),
            pl.BlockSpec((1, _H), lambda i: (0, 0)),
            pl.BlockSpec((1, _H), lambda i: (0, 0)),
        ],
        out_shape=[
            jax.ShapeDtypeStruct((_NPAD, _H), _F32),
            jax.ShapeDtypeStruct((1, _H), _F32),
            jax.ShapeDtypeStruct((1, _H), _F32),
        ],
    )(x, sums, cnts, ub, wnx, wne, wnu, bn1)


def _nout_body(hn_ref, s1_ref, s2_ref, gn_ref, bbn_ref, wn2_ref, bn2_ref,
               oh_ref, s_ref, c_ref,
               xout_ref, xg_ref, nc_ref, eg_ref, ec_ref):
    i = pl.program_id(0)
    m = s1_ref[...] * (1.0 / _N)
    v = s2_ref[...] * (1.0 / _N) - m * m
    scale = lax.rsqrt(v + _EPS) * gn_ref[...]
    hn = (hn_ref[...] - m) * scale + bbn_ref[...]
    r = jnp.maximum(hn, 0.0)
    xo = jnp.dot(r, wn2_ref[...], preferred_element_type=_F32) + bn2_ref[...]
    xout_ref[...] = xo

    oh = oh_ref[...]
    st = s_ref[0] + s_ref[1]
    c1 = jnp.sum(c_ref[0] + c_ref[1], axis=0, keepdims=True)
    crep = jnp.broadcast_to(c1, (_DE, _NB))
    dn = (((0,), (0,)), ((), ()))
    dnt = (((0,), (1,)), ((), ()))

    @pl.when(i == 0)
    def _init():
        xg_ref[...] = jnp.zeros_like(xg_ref)
        nc_ref[...] = jnp.zeros_like(nc_ref)
        eg_ref[...] = jnp.zeros_like(eg_ref)
        ec_ref[...] = jnp.zeros_like(ec_ref)

    xg_ref[...] += lax.dot_general(oh, xo, dn, preferred_element_type=_F32)
    nc_ref[...] += lax.dot_general(oh, jnp.ones((_NB, _B), _F32), dn,
                                   preferred_element_type=_F32)
    eg_ref[...] += lax.dot_general(oh, st, dnt, preferred_element_type=_F32)
    ec_ref[...] += lax.dot_general(oh, crep, dnt, preferred_element_type=_F32)


def _node_out(hn, s1, s2, gn, bbn, wn2, bn2, oh, sums, cnts):
    nblk = _NPAD // _NB
    return pl.pallas_call(
        _nout_body,
        grid=(nblk,),
        in_specs=[
            pl.BlockSpec((_NB, _H), lambda i: (i, 0)),
            pl.BlockSpec((1, _H), lambda i: (0, 0)),
            pl.BlockSpec((1, _H), lambda i: (0, 0)),
            pl.BlockSpec((1, _H), lambda i: (0, 0)),
            pl.BlockSpec((1, _H), lambda i: (0, 0)),
            pl.BlockSpec((_H, _DN), lambda i: (0, 0)),
            pl.BlockSpec((1, _DN), lambda i: (0, 0)),
            pl.BlockSpec((_NB, _B), lambda i: (i, 0)),
            pl.BlockSpec((2, _DE, _NB), lambda i: (0, 0, i)),
            pl.BlockSpec((2, _DE, _NB), lambda i: (0, 0, i)),
        ],
        out_specs=[
            pl.BlockSpec((_NB, _DN), lambda i: (i, 0)),
            pl.BlockSpec((_B, _DN), lambda i: (0, 0)),
            pl.BlockSpec((_B, _B), lambda i: (0, 0)),
            pl.BlockSpec((_B, _DE), lambda i: (0, 0)),
            pl.BlockSpec((_B, _DE), lambda i: (0, 0)),
        ],
        out_shape=[
            jax.ShapeDtypeStruct((_NPAD, _DN), _F32),
            jax.ShapeDtypeStruct((_B, _DN), _F32),
            jax.ShapeDtypeStruct((_B, _B), _F32),
            jax.ShapeDtypeStruct((_B, _DE), _F32),
            jax.ShapeDtypeStruct((_B, _DE), _F32),
        ],
    )(hn, s1, s2, gn, bbn, wn2, bn2, oh, sums, cnts)


def _glob_body(u_ref, xg_ref, nc_ref, eg_ref, ec_ref, wgu_ref, wgx_ref,
               wge_ref, bg1_ref, gg_ref, bbg_ref, wg2_ref, bg2_ref, out_ref):
    ncnt = jnp.maximum(nc_ref[...][:, 0:1], 1.0)
    xag = xg_ref[...] / ncnt
    ecnt = jnp.maximum(ec_ref[...][:, 0:1], 1.0)
    eag = eg_ref[...] / ecnt
    h = (jnp.dot(u_ref[...], wgu_ref[...], preferred_element_type=_F32)
         + jnp.dot(xag, wgx_ref[...], preferred_element_type=_F32)
         + jnp.dot(eag, wge_ref[...], preferred_element_type=_F32)
         + bg1_ref[...])
    m = jnp.mean(h, axis=0, keepdims=True)
    v = jnp.mean((h - m) * (h - m), axis=0, keepdims=True)
    hn = (h - m) * (lax.rsqrt(v + _EPS) * gg_ref[...]) + bbg_ref[...]
    r = jnp.maximum(hn, 0.0)
    out_ref[...] = jnp.dot(r, wg2_ref[...], preferred_element_type=_F32) \
        + bg2_ref[...]


def _global_out(u, xg, nc, eg, ec, wgu, wgx, wge, bg1, gg, bbg, wg2, bg2):
    full = lambda s: pl.BlockSpec(s, lambda: tuple(0 for _ in s))
    return pl.pallas_call(
        _glob_body,
        in_specs=[
            full((_B, _DG)), full((_B, _DN)), full((_B, _B)),
            full((_B, _DE)), full((_B, _DE)),
            full((_DG, _H)), full((_DN, _H)), full((_DE, _H)),
            full((1, _H)), full((1, _H)), full((1, _H)),
            full((_H, _DG)), full((1, _DG)),
        ],
        out_specs=full((_B, _DG)),
        out_shape=jax.ShapeDtypeStruct((_B, _DG), _F32),
    )(u, xg, nc, eg, ec, wgu, wgx, wge, bg1, gg, bbg, wg2, bg2)


def kernel(x, edge_index, edge_attr, u, batch,
           We1, be1, ge, bbe, We2, be2,
           Wn1, bn1, gn, bbn, Wn2, bn2,
           Wg1, bg1, gg, bbg, Wg2, bg2):
    row = edge_index[0]
    col = edge_index[1]
    xp = jnp.pad(x, ((0, _NPAD - _N), (0, 0)))
    batchp = jnp.pad(batch, (0, _NPAD - _N), constant_values=-1)
    oh = (batchp[:, None] == jnp.arange(_B, dtype=batch.dtype)[None, :]
          ).astype(_F32)

    w1s = We1[:_DN]
    w1d = We1[_DN:2 * _DN]
    w1e = We1[2 * _DN:2 * _DN + _DE]
    w1u = We1[2 * _DN + _DE:]
    ev, od = slice(0, _H, 2), slice(1, _H, 2)
    wnx = Wn1[:_DN]
    wne = Wn1[_DN:_DN + _DE]
    wnu = Wn1[_DN + _DE:]
    wgu = Wg1[:_DG]
    wgx = Wg1[_DG:_DG + _DN]
    wge = Wg1[_DG + _DN:]

    p1, p2, ub = _node_projections(
        xp, oh, u, w1s[:, ev], w1s[:, od], w1d[:, ev], w1d[:, od],
        w1u[:, ev], w1u[:, od], be1[None, ev], be1[None, od])
    g = _sc_gather_combine(p1, p2, row, col)
    s1e, s2e, s1o, s2o = _edge_stats(g, edge_attr, w1e[:, ev], w1e[:, od])
    e_out, e_outt = _edge_out(
        g, edge_attr, w1e[:, ev], w1e[:, od], s1e, s2e, s1o, s2o,
        ge[None, ev], ge[None, od], bbe[None, ev], bbe[None, od],
        We2[ev], We2[od], be2[None, :], be2[:, None])
    sums, cnts = _sc_scatter_mean(e_outt, col)
    hn, t1, t2 = _node_stats(xp, sums, cnts, ub, wnx, wne, wnu, bn1[None, :])
    x_out, xg, nc, eg, ec = _node_out(hn, t1, t2, gn[None, :], bbn[None, :],
                                      Wn2, bn2[None, :], oh, sums, cnts)
    u_out = _global_out(u, xg, nc, eg, ec, wgu, wgx, wge, bg1[None, :],
                        gg[None, :], bbg[None, :], Wg2, bg2[None, :])
    return x_out[:_N], e_out, u_out
